# baseline (speedup 1.0000x reference)
import functools

import jax
import jax.numpy as jnp
from jax import lax
from jax.experimental import pallas as pl
from jax.experimental.pallas import tpu as pltpu
from jax.experimental.pallas import tpu_sc as plsc

_E = 9
_EP = 16
_K = 2
_D = 1024
_DFF = 2048
_NTOK = 4096
_TM = 128
_NT = 74
_MPAD = _TM * _NT
_NW = 32
_RCHUNK = 1024


def _router_body(x_ref, w_ref, e1_ref, e2_ref, w1_ref, w2_ref, cts_ref, loss_ref):
    step = pl.program_id(0)
    x = x_ref[...]
    logits = lax.dot_general(x, w_ref[...], (((1,), (1,)), ((), ())),
                             preferred_element_type=jnp.float32)
    col = lax.broadcasted_iota(jnp.int32, logits.shape, 1)
    valid = col < _E
    logits = jnp.where(valid, logits, jnp.float32(-1e30))
    m = jnp.max(logits, axis=1, keepdims=True)
    ex = jnp.where(valid, jnp.exp(logits - m), 0.0)
    probs = ex / jnp.sum(ex, axis=1, keepdims=True)
    p1 = jnp.max(probs, axis=1, keepdims=True)
    e1 = jnp.min(jnp.where(probs == p1, col, _EP), axis=1, keepdims=True)
    probs2 = jnp.where(col == e1, jnp.float32(-1.0), probs)
    p2 = jnp.max(probs2, axis=1, keepdims=True)
    e2 = jnp.min(jnp.where(probs2 == p2, col, _EP), axis=1, keepdims=True)
    d = jnp.exp(p2 - p1)
    w1_ref[...] = 1.0 / (1.0 + d)
    w2_ref[...] = d / (1.0 + d)
    e1_ref[...] = e1
    e2_ref[...] = e2
    oh = (col == e1).astype(jnp.float32) + (col == e2).astype(jnp.float32)
    c = jnp.sum(oh, axis=0, keepdims=True)

    @pl.when(step == 0)
    def _():
        cts_ref[...] = c

    @pl.when(step > 0)
    def _():
        cts_ref[...] += c

    @pl.when(step == pl.num_programs(0) - 1)
    def _():
        cts = cts_ref[...]
        target = jnp.float32(_NTOK * _K / _E)
        ccol = lax.broadcasted_iota(jnp.int32, cts.shape, 1)
        sq = jnp.where(ccol < _E, (cts - target) ** 2, 0.0)
        loss_ref[...] = jnp.sum(sq, axis=1, keepdims=True) / (_E * target * target)


def _router_call(x2d, rwp):
    nsteps = _NTOK // _RCHUNK
    return pl.pallas_call(
        _router_body,
        grid=(nsteps,),
        in_specs=[
            pl.BlockSpec((_RCHUNK, _D), lambda i: (i, 0)),
            pl.BlockSpec((_EP, _D), lambda i: (0, 0)),
        ],
        out_specs=[
            pl.BlockSpec((_RCHUNK, 1), lambda i: (i, 0)),
            pl.BlockSpec((_RCHUNK, 1), lambda i: (i, 0)),
            pl.BlockSpec((_RCHUNK, 1), lambda i: (i, 0)),
            pl.BlockSpec((_RCHUNK, 1), lambda i: (i, 0)),
            pl.BlockSpec((1, _EP), lambda i: (0, 0)),
            pl.BlockSpec((1, 1), lambda i: (0, 0)),
        ],
        out_shape=[
            jax.ShapeDtypeStruct((_NTOK, 1), jnp.int32),
            jax.ShapeDtypeStruct((_NTOK, 1), jnp.int32),
            jax.ShapeDtypeStruct((_NTOK, 1), jnp.float32),
            jax.ShapeDtypeStruct((_NTOK, 1), jnp.float32),
            jax.ShapeDtypeStruct((1, _EP), jnp.float32),
            jax.ShapeDtypeStruct((1, 1), jnp.float32),
        ],
    )(x2d, rwp)


def _mlp_body(te_ref, xs_ref, gw_ref, uw_ref, dw_ref, ws_ref, out_ref):
    xs = xs_ref[...]
    g = lax.dot_general(xs, gw_ref[0], (((1,), (1,)), ((), ())),
                        preferred_element_type=jnp.float32)
    u = lax.dot_general(xs, uw_ref[0], (((1,), (1,)), ((), ())),
                        preferred_element_type=jnp.float32)
    h = (g / (1.0 + jnp.exp(-g))) * u
    o = lax.dot_general(h, dw_ref[0], (((1,), (1,)), ((), ())),
                        preferred_element_type=jnp.float32)
    out_ref[...] = o * ws_ref[...]


def _mlp_call(tile_e, xs, gate_W, up_W, down_W, wsort):
    grid_spec = pltpu.PrefetchScalarGridSpec(
        num_scalar_prefetch=1,
        grid=(_NT,),
        in_specs=[
            pl.BlockSpec((_TM, _D), lambda m, te: (m, 0)),
            pl.BlockSpec((1, _DFF, _D), lambda m, te: (te[m], 0, 0)),
            pl.BlockSpec((1, _DFF, _D), lambda m, te: (te[m], 0, 0)),
            pl.BlockSpec((1, _D, _DFF), lambda m, te: (te[m], 0, 0)),
            pl.BlockSpec((_TM, 1), lambda m, te: (m, 0)),
        ],
        out_specs=pl.BlockSpec((_TM, _D), lambda m, te: (m, 0)),
    )
    return pl.pallas_call(
        _mlp_body,
        grid_spec=grid_spec,
        out_shape=jax.ShapeDtypeStruct((_MPAD, _D), jnp.float32),
        compiler_params=pltpu.CompilerParams(vmem_limit_bytes=110 * 1024 * 1024),
    )(tile_e, xs, gate_W, up_W, down_W, wsort)


def _sc_gather_call(src, idx, n_out):
    rw = n_out // _NW
    ch = 8
    nch = rw // ch
    mesh = plsc.VectorSubcoreMesh(core_axis_name="c", subcore_axis_name="s")

    @functools.partial(
        pl.kernel,
        out_type=jax.ShapeDtypeStruct((n_out, _D), jnp.float32),
        mesh=mesh,
        scratch_types=[
            pltpu.VMEM((rw,), jnp.int32),
            pltpu.VMEM((ch, _D), jnp.float32),
            pltpu.SemaphoreType.DMA,
        ],
    )
    def k(src_hbm, idx_hbm, out_hbm, idx_v, buf_v, sem):
        wid = lax.axis_index("s") * 2 + lax.axis_index("c")
        base = wid * rw
        pltpu.sync_copy(idx_hbm.at[pl.ds(base, rw)], idx_v)

        def body(c, carry):
            pltpu.async_copy(src_hbm.at[idx_v.at[pl.ds(c * ch, ch)]], buf_v, sem).wait()
            pltpu.sync_copy(buf_v, out_hbm.at[pl.ds(base + c * ch, ch)])
            return carry

        lax.fori_loop(0, nch, body, 0)

    return k(src, idx)


def _add_body(a_ref, b_ref, o_ref):
    o_ref[...] = a_ref[...] + b_ref[...]


def _add_call(pair_rows):
    return pl.pallas_call(
        _add_body,
        grid=(4,),
        in_specs=[
            pl.BlockSpec((_RCHUNK, _D), lambda i: (i, 0)),
            pl.BlockSpec((_RCHUNK, _D), lambda i: (i + _NTOK // _RCHUNK, 0)),
        ],
        out_specs=pl.BlockSpec((_RCHUNK, _D), lambda i: (i, 0)),
        out_shape=jax.ShapeDtypeStruct((_NTOK, _D), jnp.float32),
    )(pair_rows, pair_rows)


def kernel(x, router_W, gate_W, up_W, down_W):
    x2d = x.reshape(_NTOK, _D)
    rwp = jnp.zeros((_EP, _D), jnp.float32).at[:_E].set(router_W)

    e1, e2, w1, w2, cts, loss = _router_call(x2d, rwp)
    e1, e2 = e1[:, 0], e2[:, 0]
    w_flat = jnp.concatenate([w1[:, 0], w2[:, 0]])
    e_flat = jnp.concatenate([e1, e2])
    tok = jnp.tile(jnp.arange(_NTOK, dtype=jnp.int32), _K)

    cts_i = cts[0, :_E].astype(jnp.int32)
    tiles_e = (cts_i + _TM - 1) // _TM
    cum_tiles = jnp.cumsum(tiles_e)
    row_off = _TM * jnp.concatenate([jnp.zeros((1,), jnp.int32), cum_tiles[:-1]])
    oh = (e_flat[:, None] == jnp.arange(_E, dtype=jnp.int32)[None, :]).astype(jnp.int32)
    rank = jnp.take_along_axis(jnp.cumsum(oh, axis=0) - oh, e_flat[:, None], axis=1)[:, 0]
    pos = row_off[e_flat] + rank
    gidx = jnp.zeros((_MPAD,), jnp.int32).at[pos].set(tok)
    wsort = jnp.zeros((_MPAD,), jnp.float32).at[pos].set(w_flat)
    tile_e = jnp.minimum(
        jnp.searchsorted(cum_tiles, jnp.arange(_NT, dtype=jnp.int32), side="right"),
        _E - 1,
    ).astype(jnp.int32)

    xs = _sc_gather_call(x2d, gidx, _MPAD)
    outs = _mlp_call(tile_e, xs, gate_W, up_W, down_W, wsort[:, None])
    final = outs[:_NTOK] + pos[:_NTOK, None]
    return final.reshape(x.shape), loss[0, 0]


# speedup vs baseline: 1.1613x; 1.1613x over previous
import functools

import jax
import jax.numpy as jnp
from jax import lax
from jax.experimental import pallas as pl
from jax.experimental.pallas import tpu as pltpu
from jax.experimental.pallas import tpu_sc as plsc

_E = 9
_EP = 16
_K = 2
_D = 1024
_DFF = 2048
_NTOK = 4096
_NPAIR = _NTOK * _K
_TM = 128
_NT = 74
_MPAD = _TM * _NT
_NW = 32
_RCHUNK = 1024


def _router_body(x_ref, w_ref, e1_ref, e2_ref, w1_ref, w2_ref, r1_ref, r2_ref,
                 cts_ref, loss_ref):
    step = pl.program_id(0)
    x = x_ref[...]
    logits = lax.dot_general(x, w_ref[...], (((1,), (1,)), ((), ())),
                             preferred_element_type=jnp.float32)
    col = lax.broadcasted_iota(jnp.int32, logits.shape, 1)
    valid = col < _E
    logits = jnp.where(valid, logits, jnp.float32(-1e30))
    m = jnp.max(logits, axis=1, keepdims=True)
    ex = jnp.where(valid, jnp.exp(logits - m), 0.0)
    probs = ex / jnp.sum(ex, axis=1, keepdims=True)
    p1 = jnp.max(probs, axis=1, keepdims=True)
    e1 = jnp.min(jnp.where(probs == p1, col, _EP), axis=1, keepdims=True)
    probs2 = jnp.where(col == e1, jnp.float32(-1.0), probs)
    p2 = jnp.max(probs2, axis=1, keepdims=True)
    e2 = jnp.min(jnp.where(probs2 == p2, col, _EP), axis=1, keepdims=True)
    d = jnp.exp(p2 - p1)
    w1_ref[...] = 1.0 / (1.0 + d)
    w2_ref[...] = d / (1.0 + d)
    e1_ref[...] = e1
    e2_ref[...] = e2

    oh = (col == e1).astype(jnp.float32) + (col == e2).astype(jnp.float32)
    rr = lax.broadcasted_iota(jnp.int32, (_RCHUNK, _RCHUNK), 0)
    cc = lax.broadcasted_iota(jnp.int32, (_RCHUNK, _RCHUNK), 1)
    tri = (cc < rr).astype(jnp.float32)
    pref = lax.dot_general(tri, oh, (((1,), (0,)), ((), ())),
                           preferred_element_type=jnp.float32)
    base = jnp.where(step == 0, jnp.zeros_like(cts_ref[...]), cts_ref[...])
    run = base + pref
    rank1 = jnp.sum(jnp.where(col == e1, run, 0.0), axis=1, keepdims=True)
    rank2 = jnp.sum(jnp.where(col == e2, run, 0.0), axis=1, keepdims=True)
    r1_ref[...] = rank1.astype(jnp.int32)
    r2_ref[...] = rank2.astype(jnp.int32)
    cts = base + jnp.sum(oh, axis=0, keepdims=True)
    cts_ref[...] = cts

    @pl.when(step == pl.num_programs(0) - 1)
    def _():
        target = jnp.float32(_NTOK * _K / _E)
        ccol = lax.broadcasted_iota(jnp.int32, cts.shape, 1)
        sq = jnp.where(ccol < _E, (cts - target) ** 2, 0.0)
        loss_ref[...] = jnp.sum(sq, axis=1, keepdims=True) / (_E * target * target)


def _router_call(x2d, rwp):
    nsteps = _NTOK // _RCHUNK
    colspec = pl.BlockSpec((_RCHUNK, 1), lambda i: (i, 0))
    return pl.pallas_call(
        _router_body,
        grid=(nsteps,),
        in_specs=[
            pl.BlockSpec((_RCHUNK, _D), lambda i: (i, 0)),
            pl.BlockSpec((_EP, _D), lambda i: (0, 0)),
        ],
        out_specs=[
            colspec, colspec, colspec, colspec, colspec, colspec,
            pl.BlockSpec((1, _EP), lambda i: (0, 0)),
            pl.BlockSpec((1, 1), lambda i: (0, 0)),
        ],
        out_shape=[
            jax.ShapeDtypeStruct((_NTOK, 1), jnp.int32),
            jax.ShapeDtypeStruct((_NTOK, 1), jnp.int32),
            jax.ShapeDtypeStruct((_NTOK, 1), jnp.float32),
            jax.ShapeDtypeStruct((_NTOK, 1), jnp.float32),
            jax.ShapeDtypeStruct((_NTOK, 1), jnp.int32),
            jax.ShapeDtypeStruct((_NTOK, 1), jnp.int32),
            jax.ShapeDtypeStruct((1, _EP), jnp.float32),
            jax.ShapeDtypeStruct((1, 1), jnp.float32),
        ],
    )(x2d, rwp)


def _mlp_body(te_ref, xs_ref, gw_ref, uw_ref, dw_ref, out_ref):
    xs = xs_ref[...]
    g = lax.dot_general(xs, gw_ref[0], (((1,), (1,)), ((), ())),
                        preferred_element_type=jnp.float32)
    u = lax.dot_general(xs, uw_ref[0], (((1,), (1,)), ((), ())),
                        preferred_element_type=jnp.float32)
    h = (g / (1.0 + jnp.exp(-g))) * u
    o = lax.dot_general(h, dw_ref[0], (((1,), (1,)), ((), ())),
                        preferred_element_type=jnp.float32)
    out_ref[...] = o


def _mlp_call(tile_e, xs, gate_W, up_W, down_W):
    grid_spec = pltpu.PrefetchScalarGridSpec(
        num_scalar_prefetch=1,
        grid=(_NT,),
        in_specs=[
            pl.BlockSpec((_TM, _D), lambda m, te: (m, 0)),
            pl.BlockSpec((1, _DFF, _D), lambda m, te: (te[m], 0, 0)),
            pl.BlockSpec((1, _DFF, _D), lambda m, te: (te[m], 0, 0)),
            pl.BlockSpec((1, _D, _DFF), lambda m, te: (te[m], 0, 0)),
        ],
        out_specs=pl.BlockSpec((_TM, _D), lambda m, te: (m, 0)),
    )
    return pl.pallas_call(
        _mlp_body,
        grid_spec=grid_spec,
        out_shape=jax.ShapeDtypeStruct((_MPAD, _D), jnp.float32),
        compiler_params=pltpu.CompilerParams(vmem_limit_bytes=110 * 1024 * 1024),
    )(tile_e, xs, gate_W, up_W, down_W)


def _sc_scatter_call(x2d, pos2d):
    ppw = _NPAIR // _NW
    ch = 64
    nch = ppw // ch
    mesh = plsc.VectorSubcoreMesh(core_axis_name="c", subcore_axis_name="s")

    @functools.partial(
        pl.kernel,
        out_type=jax.ShapeDtypeStruct((_MPAD, _D), jnp.float32),
        mesh=mesh,
        scratch_types=[
            pltpu.VMEM((nch, ch), jnp.int32),
            pltpu.VMEM((ch, _D), jnp.float32),
            pltpu.SemaphoreType.DMA,
        ],
    )
    def k(x_hbm, pos_hbm, out_hbm, pos_v, buf_v, sem):
        wid = lax.axis_index("s") * 2 + lax.axis_index("c")
        tbase = (wid % (_NW // _K)) * ppw
        pltpu.sync_copy(pos_hbm.at[pl.ds(wid * nch, nch)], pos_v)
        for c in range(nch):
            pltpu.sync_copy(x_hbm.at[pl.ds(tbase + c * ch, ch)], buf_v)
            pltpu.async_copy(buf_v, out_hbm.at[pos_v.at[c]], sem).wait()

    return k(x2d, pos2d)


def _sc_gather_call(src, idx):
    rw = _NPAIR // _NW
    ch = 32
    nch = rw // ch
    mesh = plsc.VectorSubcoreMesh(core_axis_name="c", subcore_axis_name="s")

    @functools.partial(
        pl.kernel,
        out_type=jax.ShapeDtypeStruct((_NPAIR, _D), jnp.float32),
        mesh=mesh,
        scratch_types=[
            pltpu.VMEM((rw,), jnp.int32),
            pltpu.VMEM((ch, _D), jnp.float32),
            pltpu.SemaphoreType.DMA,
        ],
    )
    def k(src_hbm, idx_hbm, out_hbm, idx_v, buf_v, sem):
        wid = lax.axis_index("s") * 2 + lax.axis_index("c")
        base = wid * rw
        pltpu.sync_copy(idx_hbm.at[pl.ds(base, rw)], idx_v)

        def body(c, carry):
            pltpu.async_copy(src_hbm.at[idx_v.at[pl.ds(c * ch, ch)]], buf_v, sem).wait()
            pltpu.sync_copy(buf_v, out_hbm.at[pl.ds(base + c * ch, ch)])
            return carry

        lax.fori_loop(0, nch, body, 0)

    return k(src, idx)


def _combine_body(a_ref, b_ref, w1_ref, w2_ref, o_ref):
    o_ref[...] = a_ref[...] * w1_ref[...] + b_ref[...] * w2_ref[...]


def _combine_call(pair_rows, w1, w2):
    nb = _NTOK // _RCHUNK
    return pl.pallas_call(
        _combine_body,
        grid=(nb,),
        in_specs=[
            pl.BlockSpec((_RCHUNK, _D), lambda i: (i, 0)),
            pl.BlockSpec((_RCHUNK, _D), lambda i: (i + nb, 0)),
            pl.BlockSpec((_RCHUNK, 1), lambda i: (i, 0)),
            pl.BlockSpec((_RCHUNK, 1), lambda i: (i, 0)),
        ],
        out_specs=pl.BlockSpec((_RCHUNK, _D), lambda i: (i, 0)),
        out_shape=jax.ShapeDtypeStruct((_NTOK, _D), jnp.float32),
    )(pair_rows, pair_rows, w1, w2)


def kernel(x, router_W, gate_W, up_W, down_W):
    x2d = x.reshape(_NTOK, _D)
    rwp = jnp.zeros((_EP, _D), jnp.float32).at[:_E].set(router_W)

    e1, e2, w1, w2, r1, r2, cts, loss = _router_call(x2d, rwp)

    cts_i = cts[0, :_E].astype(jnp.int32)
    tiles_e = (cts_i + _TM - 1) // _TM
    cum_tiles = jnp.cumsum(tiles_e)
    row_off = _TM * (cum_tiles - tiles_e)
    pos1 = row_off[e1[:, 0]] + r1[:, 0]
    pos2 = row_off[e2[:, 0]] + r2[:, 0]
    pos = jnp.concatenate([pos1, pos2])
    tile_e = jnp.minimum(
        jnp.searchsorted(cum_tiles, jnp.arange(_NT, dtype=jnp.int32), side="right"),
        _E - 1,
    ).astype(jnp.int32)

    xs = _sc_scatter_call(x2d, pos.reshape(_NPAIR // 64, 64))
    outs = _mlp_call(tile_e, xs, gate_W, up_W, down_W)
    pair_rows = _sc_gather_call(outs, pos)
    final = _combine_call(pair_rows, w1, w2)
    return final.reshape(x.shape), loss[0, 0]


# speedup vs baseline: 1.7657x; 1.5205x over previous
import functools

import jax
import jax.numpy as jnp
from jax import lax
from jax.experimental import pallas as pl
from jax.experimental.pallas import tpu as pltpu
from jax.experimental.pallas import tpu_sc as plsc

_E = 9
_EP = 16
_K = 2
_D = 1024
_DFF = 2048
_NTOK = 4096
_NPAIR = _NTOK * _K
_TM = 256
_NT = 41
_MPAD = _TM * _NT
_NW = 32
_RCHUNK = 1024


def _router_body(x_ref, w_ref, e1_ref, e2_ref, w1_ref, w2_ref, r1_ref, r2_ref,
                 cts_ref, loss_ref):
    step = pl.program_id(0)
    x = x_ref[...]
    logits = lax.dot_general(x, w_ref[...], (((1,), (1,)), ((), ())),
                             preferred_element_type=jnp.float32)
    col = lax.broadcasted_iota(jnp.int32, logits.shape, 1)
    valid = col < _E
    logits = jnp.where(valid, logits, jnp.float32(-1e30))
    m = jnp.max(logits, axis=1, keepdims=True)
    ex = jnp.where(valid, jnp.exp(logits - m), 0.0)
    probs = ex / jnp.sum(ex, axis=1, keepdims=True)
    p1 = jnp.max(probs, axis=1, keepdims=True)
    e1 = jnp.min(jnp.where(probs == p1, col, _EP), axis=1, keepdims=True)
    probs2 = jnp.where(col == e1, jnp.float32(-1.0), probs)
    p2 = jnp.max(probs2, axis=1, keepdims=True)
    e2 = jnp.min(jnp.where(probs2 == p2, col, _EP), axis=1, keepdims=True)
    d = jnp.exp(p2 - p1)
    w1_ref[...] = 1.0 / (1.0 + d)
    w2_ref[...] = d / (1.0 + d)
    e1_ref[...] = e1
    e2_ref[...] = e2

    oh = (col == e1).astype(jnp.float32) + (col == e2).astype(jnp.float32)
    rr = lax.broadcasted_iota(jnp.int32, (_RCHUNK, _RCHUNK), 0)
    cc = lax.broadcasted_iota(jnp.int32, (_RCHUNK, _RCHUNK), 1)
    tri = (cc < rr).astype(jnp.float32)
    pref = lax.dot_general(tri, oh, (((1,), (0,)), ((), ())),
                           preferred_element_type=jnp.float32)
    base = jnp.where(step == 0, jnp.zeros_like(cts_ref[...]), cts_ref[...])
    run = base + pref
    rank1 = jnp.sum(jnp.where(col == e1, run, 0.0), axis=1, keepdims=True)
    rank2 = jnp.sum(jnp.where(col == e2, run, 0.0), axis=1, keepdims=True)
    r1_ref[...] = rank1.astype(jnp.int32)
    r2_ref[...] = rank2.astype(jnp.int32)
    cts = base + jnp.sum(oh, axis=0, keepdims=True)
    cts_ref[...] = cts

    @pl.when(step == pl.num_programs(0) - 1)
    def _():
        target = jnp.float32(_NTOK * _K / _E)
        ccol = lax.broadcasted_iota(jnp.int32, cts.shape, 1)
        sq = jnp.where(ccol < _E, (cts - target) ** 2, 0.0)
        loss_ref[...] = jnp.sum(sq, axis=1, keepdims=True) / (_E * target * target)


def _router_call(x2d, rwp):
    nsteps = _NTOK // _RCHUNK
    colspec = pl.BlockSpec((_RCHUNK, 1), lambda i: (i, 0))
    return pl.pallas_call(
        _router_body,
        grid=(nsteps,),
        in_specs=[
            pl.BlockSpec((_RCHUNK, _D), lambda i: (i, 0)),
            pl.BlockSpec((_EP, _D), lambda i: (0, 0)),
        ],
        out_specs=[
            colspec, colspec, colspec, colspec, colspec, colspec,
            pl.BlockSpec((1, _EP), lambda i: (0, 0)),
            pl.BlockSpec((1, 1), lambda i: (0, 0)),
        ],
        out_shape=[
            jax.ShapeDtypeStruct((_NTOK, 1), jnp.int32),
            jax.ShapeDtypeStruct((_NTOK, 1), jnp.int32),
            jax.ShapeDtypeStruct((_NTOK, 1), jnp.float32),
            jax.ShapeDtypeStruct((_NTOK, 1), jnp.float32),
            jax.ShapeDtypeStruct((_NTOK, 1), jnp.int32),
            jax.ShapeDtypeStruct((_NTOK, 1), jnp.int32),
            jax.ShapeDtypeStruct((1, _EP), jnp.float32),
            jax.ShapeDtypeStruct((1, 1), jnp.float32),
        ],
    )(x2d, rwp)


def _mlp_body(te_ref, xs_ref, gw_ref, uw_ref, dw_ref, out_ref):
    xs = xs_ref[...]
    g = lax.dot_general(xs, gw_ref[0], (((1,), (1,)), ((), ())),
                        preferred_element_type=jnp.float32)
    u = lax.dot_general(xs, uw_ref[0], (((1,), (1,)), ((), ())),
                        preferred_element_type=jnp.float32)
    h = (g / (1.0 + jnp.exp(-g))) * u
    o = lax.dot_general(h, dw_ref[0], (((1,), (1,)), ((), ())),
                        preferred_element_type=jnp.float32)
    out_ref[...] = o


def _mlp_call(tile_e, xs, gate_W, up_W, down_W):
    grid_spec = pltpu.PrefetchScalarGridSpec(
        num_scalar_prefetch=1,
        grid=(_NT,),
        in_specs=[
            pl.BlockSpec((_TM, _D), lambda m, te: (m, 0)),
            pl.BlockSpec((1, _DFF, _D), lambda m, te: (te[m], 0, 0)),
            pl.BlockSpec((1, _DFF, _D), lambda m, te: (te[m], 0, 0)),
            pl.BlockSpec((1, _D, _DFF), lambda m, te: (te[m], 0, 0)),
        ],
        out_specs=pl.BlockSpec((_TM, _D), lambda m, te: (m, 0)),
    )
    return pl.pallas_call(
        _mlp_body,
        grid_spec=grid_spec,
        out_shape=jax.ShapeDtypeStruct((_MPAD, _D), jnp.float32),
        compiler_params=pltpu.CompilerParams(vmem_limit_bytes=110 * 1024 * 1024),
    )(tile_e, xs, gate_W, up_W, down_W)


def _sc_scatter_call(x2d, pos2d):
    ppw = _NPAIR // _NW
    ch = 64
    nch = ppw // ch
    mesh = plsc.VectorSubcoreMesh(core_axis_name="c", subcore_axis_name="s")

    @functools.partial(
        pl.kernel,
        out_type=jax.ShapeDtypeStruct((_MPAD, _D), jnp.float32),
        mesh=mesh,
        scratch_types=[
            pltpu.VMEM((nch, ch), jnp.int32),
            pltpu.VMEM((ch, _D), jnp.float32),
            pltpu.SemaphoreType.DMA,
        ],
    )
    def k(x_hbm, pos_hbm, out_hbm, pos_v, buf_v, sem):
        wid = lax.axis_index("s") * 2 + lax.axis_index("c")
        tbase = (wid % (_NW // _K)) * ppw
        pltpu.sync_copy(pos_hbm.at[pl.ds(wid * nch, nch)], pos_v)
        for c in range(nch):
            pltpu.sync_copy(x_hbm.at[pl.ds(tbase + c * ch, ch)], buf_v)
            pltpu.async_copy(buf_v, out_hbm.at[pos_v.at[c]], sem).wait()

    return k(x2d, pos2d)


def _sc_gather_call(src, idx):
    rw = _NPAIR // _NW
    ch = 32
    nch = rw // ch
    mesh = plsc.VectorSubcoreMesh(core_axis_name="c", subcore_axis_name="s")

    @functools.partial(
        pl.kernel,
        out_type=jax.ShapeDtypeStruct((_NPAIR, _D), jnp.float32),
        mesh=mesh,
        scratch_types=[
            pltpu.VMEM((rw,), jnp.int32),
            pltpu.VMEM((ch, _D), jnp.float32),
            pltpu.SemaphoreType.DMA,
        ],
    )
    def k(src_hbm, idx_hbm, out_hbm, idx_v, buf_v, sem):
        wid = lax.axis_index("s") * 2 + lax.axis_index("c")
        base = wid * rw
        pltpu.sync_copy(idx_hbm.at[pl.ds(base, rw)], idx_v)

        def body(c, carry):
            pltpu.async_copy(src_hbm.at[idx_v.at[pl.ds(c * ch, ch)]], buf_v, sem).wait()
            pltpu.sync_copy(buf_v, out_hbm.at[pl.ds(base + c * ch, ch)])
            return carry

        lax.fori_loop(0, nch, body, 0)

    return k(src, idx)


def _combine_body(a_ref, b_ref, w1_ref, w2_ref, o_ref):
    o_ref[...] = a_ref[...] * w1_ref[...] + b_ref[...] * w2_ref[...]


def _combine_call(pair_rows, w1, w2):
    nb = _NTOK // _RCHUNK
    return pl.pallas_call(
        _combine_body,
        grid=(nb,),
        in_specs=[
            pl.BlockSpec((_RCHUNK, _D), lambda i: (i, 0)),
            pl.BlockSpec((_RCHUNK, _D), lambda i: (i + nb, 0)),
            pl.BlockSpec((_RCHUNK, 1), lambda i: (i, 0)),
            pl.BlockSpec((_RCHUNK, 1), lambda i: (i, 0)),
        ],
        out_specs=pl.BlockSpec((_RCHUNK, _D), lambda i: (i, 0)),
        out_shape=jax.ShapeDtypeStruct((_NTOK, _D), jnp.float32),
    )(pair_rows, pair_rows, w1, w2)


def kernel(x, router_W, gate_W, up_W, down_W):
    x2d = x.reshape(_NTOK, _D)
    rwp = jnp.zeros((_EP, _D), jnp.float32).at[:_E].set(router_W)

    e1, e2, w1, w2, r1, r2, cts, loss = _router_call(x2d, rwp)

    cts_i = cts[0, :_E].astype(jnp.int32)
    tiles_e = (cts_i + _TM - 1) // _TM
    cum_tiles = jnp.cumsum(tiles_e)
    row_off = _TM * (cum_tiles - tiles_e)
    pos1 = row_off[e1[:, 0]] + r1[:, 0]
    pos2 = row_off[e2[:, 0]] + r2[:, 0]
    pos = jnp.concatenate([pos1, pos2])
    tile_e = jnp.minimum(
        jnp.searchsorted(cum_tiles, jnp.arange(_NT, dtype=jnp.int32), side="right"),
        _E - 1,
    ).astype(jnp.int32)

    xs = _sc_scatter_call(x2d, pos.reshape(_NPAIR // 64, 64))
    outs = _mlp_call(tile_e, xs, gate_W, up_W, down_W)
    pair_rows = _sc_gather_call(outs, pos)
    final = _combine_call(pair_rows, w1, w2)
    return final.reshape(x.shape), loss[0, 0]


# speedup vs baseline: 1.9041x; 1.0784x over previous
import functools

import jax
import jax.numpy as jnp
from jax import lax
from jax.experimental import pallas as pl
from jax.experimental.pallas import tpu as pltpu
from jax.experimental.pallas import tpu_sc as plsc

_E = 9
_EP = 16
_K = 2
_D = 1024
_DFF = 2048
_NTOK = 4096
_NPAIR = _NTOK * _K
_TM = 256
_NT = 41
_MPAD = _TM * _NT
_NW = 32
_RCHUNK = 1024


def _router_body(x_ref, w_ref, e1_ref, e2_ref, w1_ref, w2_ref, r1_ref, r2_ref,
                 cts_ref, loss_ref):
    step = pl.program_id(0)
    x = x_ref[...]
    logits = lax.dot_general(x, w_ref[...], (((1,), (1,)), ((), ())),
                             preferred_element_type=jnp.float32)
    col = lax.broadcasted_iota(jnp.int32, logits.shape, 1)
    valid = col < _E
    logits = jnp.where(valid, logits, jnp.float32(-1e30))
    m = jnp.max(logits, axis=1, keepdims=True)
    ex = jnp.where(valid, jnp.exp(logits - m), 0.0)
    probs = ex / jnp.sum(ex, axis=1, keepdims=True)
    p1 = jnp.max(probs, axis=1, keepdims=True)
    e1 = jnp.min(jnp.where(probs == p1, col, _EP), axis=1, keepdims=True)
    probs2 = jnp.where(col == e1, jnp.float32(-1.0), probs)
    p2 = jnp.max(probs2, axis=1, keepdims=True)
    e2 = jnp.min(jnp.where(probs2 == p2, col, _EP), axis=1, keepdims=True)
    d = jnp.exp(p2 - p1)
    w1_ref[...] = 1.0 / (1.0 + d)
    w2_ref[...] = d / (1.0 + d)
    e1_ref[...] = e1
    e2_ref[...] = e2

    oh = (col == e1).astype(jnp.float32) + (col == e2).astype(jnp.float32)
    rr = lax.broadcasted_iota(jnp.int32, (_RCHUNK, _RCHUNK), 0)
    cc = lax.broadcasted_iota(jnp.int32, (_RCHUNK, _RCHUNK), 1)
    tri = (cc < rr).astype(jnp.float32)
    pref = lax.dot_general(tri, oh, (((1,), (0,)), ((), ())),
                           preferred_element_type=jnp.float32)
    base = jnp.where(step == 0, jnp.zeros_like(cts_ref[...]), cts_ref[...])
    run = base + pref
    rank1 = jnp.sum(jnp.where(col == e1, run, 0.0), axis=1, keepdims=True)
    rank2 = jnp.sum(jnp.where(col == e2, run, 0.0), axis=1, keepdims=True)
    r1_ref[...] = rank1.astype(jnp.int32)
    r2_ref[...] = rank2.astype(jnp.int32)
    cts = base + jnp.sum(oh, axis=0, keepdims=True)
    cts_ref[...] = cts

    @pl.when(step == pl.num_programs(0) - 1)
    def _():
        target = jnp.float32(_NTOK * _K / _E)
        ccol = lax.broadcasted_iota(jnp.int32, cts.shape, 1)
        sq = jnp.where(ccol < _E, (cts - target) ** 2, 0.0)
        loss_ref[...] = jnp.sum(sq, axis=1, keepdims=True) / (_E * target * target)


def _router_call(x2d, rwp):
    nsteps = _NTOK // _RCHUNK
    colspec = pl.BlockSpec((_RCHUNK, 1), lambda i: (i, 0))
    return pl.pallas_call(
        _router_body,
        grid=(nsteps,),
        in_specs=[
            pl.BlockSpec((_RCHUNK, _D), lambda i: (i, 0)),
            pl.BlockSpec((_EP, _D), lambda i: (0, 0)),
        ],
        out_specs=[
            colspec, colspec, colspec, colspec, colspec, colspec,
            pl.BlockSpec((1, _EP), lambda i: (0, 0)),
            pl.BlockSpec((1, 1), lambda i: (0, 0)),
        ],
        out_shape=[
            jax.ShapeDtypeStruct((_NTOK, 1), jnp.int32),
            jax.ShapeDtypeStruct((_NTOK, 1), jnp.int32),
            jax.ShapeDtypeStruct((_NTOK, 1), jnp.float32),
            jax.ShapeDtypeStruct((_NTOK, 1), jnp.float32),
            jax.ShapeDtypeStruct((_NTOK, 1), jnp.int32),
            jax.ShapeDtypeStruct((_NTOK, 1), jnp.int32),
            jax.ShapeDtypeStruct((1, _EP), jnp.float32),
            jax.ShapeDtypeStruct((1, 1), jnp.float32),
        ],
    )(x2d, rwp)


def _mlp_body(te_ref, nu_ref, xs_ref, gw_ref, uw_ref, dw_ref, out_ref):
    m = pl.program_id(0)

    @pl.when(m < nu_ref[0])
    def _():
        xs = xs_ref[...]
        g = lax.dot_general(xs, gw_ref[0], (((1,), (1,)), ((), ())),
                            preferred_element_type=jnp.float32)
        u = lax.dot_general(xs, uw_ref[0], (((1,), (1,)), ((), ())),
                            preferred_element_type=jnp.float32)
        h = (g / (1.0 + jnp.exp(-g))) * u
        o = lax.dot_general(h, dw_ref[0], (((1,), (1,)), ((), ())),
                            preferred_element_type=jnp.float32)
        out_ref[...] = o


def _mlp_call(tile_e, nu, xs, gate_W, up_W, down_W):
    grid_spec = pltpu.PrefetchScalarGridSpec(
        num_scalar_prefetch=2,
        grid=(_NT,),
        in_specs=[
            pl.BlockSpec((_TM, _D), lambda m, te, nu: (m, 0)),
            pl.BlockSpec((1, _DFF, _D), lambda m, te, nu: (te[m], 0, 0)),
            pl.BlockSpec((1, _DFF, _D), lambda m, te, nu: (te[m], 0, 0)),
            pl.BlockSpec((1, _D, _DFF), lambda m, te, nu: (te[m], 0, 0)),
        ],
        out_specs=pl.BlockSpec((_TM, _D), lambda m, te, nu: (m, 0)),
    )
    return pl.pallas_call(
        _mlp_body,
        grid_spec=grid_spec,
        out_shape=jax.ShapeDtypeStruct((_MPAD, _D), jnp.float32),
        compiler_params=pltpu.CompilerParams(vmem_limit_bytes=110 * 1024 * 1024),
    )(tile_e, nu, xs, gate_W, up_W, down_W)


def _sc_scatter_call(x2d, e2d, r2d, cnt16):
    ppw = _NPAIR // _NW
    ch = 32
    nch = ppw // ch
    mesh = plsc.VectorSubcoreMesh(core_axis_name="c", subcore_axis_name="s")

    @functools.partial(
        pl.kernel,
        out_type=[
            jax.ShapeDtypeStruct((_MPAD, _D), jnp.float32),
            jax.ShapeDtypeStruct((_NW, nch, ch), jnp.int32),
        ],
        mesh=mesh,
        compiler_params=pltpu.CompilerParams(needs_layout_passes=False),
        scratch_types=[
            pltpu.VMEM((1, ppw), jnp.int32),
            pltpu.VMEM((1, ppw), jnp.int32),
            pltpu.VMEM((16,), jnp.int32),
            pltpu.VMEM((nch, ch), jnp.int32),
            pltpu.VMEM((2, ch, _D), jnp.float32),
            pltpu.SemaphoreType.DMA,
            pltpu.SemaphoreType.DMA,
        ],
    )
    def k(x_hbm, e_hbm, r_hbm, c_hbm, out_hbm, pos_hbm, e_v, r_v, off_v, pos_v,
          buf_v, sem0, sem1):
        wid = lax.axis_index("s") * 2 + lax.axis_index("c")
        tbase = (wid % (_NW // _K)) * ppw
        pltpu.sync_copy(e_hbm.at[pl.ds(wid, 1)], e_v)
        pltpu.sync_copy(r_hbm.at[pl.ds(wid, 1)], r_v)
        pltpu.sync_copy(c_hbm, off_v)
        cnt = off_v[...]
        tiles = jnp.right_shift(cnt + (_TM - 1), _TM.bit_length() - 1)
        off_v[...] = (plsc.cumsum(tiles) - tiles) * _TM
        for j in range(ppw // 16):
            ev = e_v[0, pl.ds(j * 16, 16)]
            rv = r_v[0, pl.ds(j * 16, 16)]
            pv = plsc.load_gather(off_v, [ev]) + rv
            pos_v[j // (ch // 16), pl.ds((j % (ch // 16)) * 16, 16)] = pv
        pltpu.sync_copy(pos_v, pos_hbm.at[wid])
        sems = (sem0, sem1)
        pltpu.sync_copy(x_hbm.at[pl.ds(tbase, ch)], buf_v.at[0])
        cps = [None, None]
        for c in range(nch):
            s = c % 2
            cps[s] = pltpu.async_copy(buf_v.at[s], out_hbm.at[pos_v.at[c]], sems[s])
            if c + 1 < nch:
                if cps[1 - s] is not None:
                    cps[1 - s].wait()
                pltpu.sync_copy(x_hbm.at[pl.ds(tbase + (c + 1) * ch, ch)],
                                buf_v.at[1 - s])
        cps[(nch - 1) % 2].wait()

    return k(x2d, e2d, r2d, cnt16)


def _sc_gather_call(src, idx3d):
    rw = _NPAIR // _NW
    ch = 32
    nch = rw // ch
    nb = 3
    mesh = plsc.VectorSubcoreMesh(core_axis_name="c", subcore_axis_name="s")

    @functools.partial(
        pl.kernel,
        out_type=jax.ShapeDtypeStruct((_NPAIR, _D), jnp.float32),
        mesh=mesh,
        scratch_types=[
            pltpu.VMEM((rw,), jnp.int32),
            pltpu.VMEM((nb, ch, _D), jnp.float32),
            [pltpu.SemaphoreType.DMA] * nb,
            [pltpu.SemaphoreType.DMA] * nb,
        ],
    )
    def k(src_hbm, idx_hbm, out_hbm, idx_v, buf_v, gsems, wsems):
        wid = lax.axis_index("s") * 2 + lax.axis_index("c")
        base = wid * rw
        pltpu.sync_copy(idx_hbm.at[wid], idx_v)
        gcps = [None] * nb
        wcps = [None] * nb
        for c in range(min(nb, nch)):
            gcps[c] = pltpu.async_copy(
                src_hbm.at[idx_v.at[pl.ds(c * ch, ch)]], buf_v.at[c], gsems[c])
        for c in range(nch):
            s = c % nb
            gcps[s].wait()
            wcps[s] = pltpu.async_copy(
                buf_v.at[s], out_hbm.at[pl.ds(base + c * ch, ch)], wsems[s])
            nxt = c + nb
            if nxt < nch:
                wcps[s].wait()
                gcps[s] = pltpu.async_copy(
                    src_hbm.at[idx_v.at[pl.ds(nxt * ch, ch)]], buf_v.at[s], gsems[s])
        for s in range(min(nb, nch)):
            if wcps[s] is not None:
                wcps[s].wait()

    return k(src, idx3d)


def _combine_body(a_ref, b_ref, w1_ref, w2_ref, o_ref):
    o_ref[...] = a_ref[...] * w1_ref[...] + b_ref[...] * w2_ref[...]


def _combine_call(pair_rows, w1, w2):
    nb = _NTOK // _RCHUNK
    return pl.pallas_call(
        _combine_body,
        grid=(nb,),
        in_specs=[
            pl.BlockSpec((_RCHUNK, _D), lambda i: (i, 0)),
            pl.BlockSpec((_RCHUNK, _D), lambda i: (i + nb, 0)),
            pl.BlockSpec((_RCHUNK, 1), lambda i: (i, 0)),
            pl.BlockSpec((_RCHUNK, 1), lambda i: (i, 0)),
        ],
        out_specs=pl.BlockSpec((_RCHUNK, _D), lambda i: (i, 0)),
        out_shape=jax.ShapeDtypeStruct((_NTOK, _D), jnp.float32),
    )(pair_rows, pair_rows, w1, w2)


def kernel(x, router_W, gate_W, up_W, down_W):
    x2d = x.reshape(_NTOK, _D)
    rwp = jnp.zeros((_EP, _D), jnp.float32).at[:_E].set(router_W)

    e1, e2, w1, w2, r1, r2, cts, loss = _router_call(x2d, rwp)

    cnt16 = cts[0].astype(jnp.int32)
    tiles_e = (cnt16[:_E] + _TM - 1) // _TM
    cum_tiles = jnp.cumsum(tiles_e)
    nu = cum_tiles[-1:]
    tile_e = jnp.minimum(
        jnp.searchsorted(cum_tiles, jnp.arange(_NT, dtype=jnp.int32), side="right"),
        _E - 1,
    ).astype(jnp.int32)

    ppw = _NPAIR // _NW
    e2d = jnp.concatenate([e1[:, 0], e2[:, 0]]).reshape(_NW, ppw)
    r2d = jnp.concatenate([r1[:, 0], r2[:, 0]]).reshape(_NW, ppw)

    xs, pos3 = _sc_scatter_call(x2d, e2d, r2d, cnt16)
    outs = _mlp_call(tile_e, nu, xs, gate_W, up_W, down_W)
    pair_rows = _sc_gather_call(outs, pos3.reshape(_NW, ppw))
    final = _combine_call(pair_rows, w1, w2)
    return final.reshape(x.shape), loss[0, 0]


# speedup vs baseline: 2.0344x; 1.0684x over previous
import functools

import jax
import jax.numpy as jnp
from jax import lax
from jax.experimental import pallas as pl
from jax.experimental.pallas import tpu as pltpu
from jax.experimental.pallas import tpu_sc as plsc

_E = 9
_EP = 16
_K = 2
_D = 1024
_DFF = 2048
_NTOK = 4096
_NPAIR = _NTOK * _K
_TM = 512
_NT = 25
_MPAD = _TM * _NT
_NW = 32
_RCHUNK = 1024


def _router_body(x_ref, w_ref, e1_ref, e2_ref, w1_ref, w2_ref, r1_ref, r2_ref,
                 cts_ref, loss_ref):
    step = pl.program_id(0)
    x = x_ref[...]
    logits = lax.dot_general(x, w_ref[...], (((1,), (1,)), ((), ())),
                             preferred_element_type=jnp.float32)
    col = lax.broadcasted_iota(jnp.int32, logits.shape, 1)
    valid = col < _E
    logits = jnp.where(valid, logits, jnp.float32(-1e30))
    m = jnp.max(logits, axis=1, keepdims=True)
    ex = jnp.where(valid, jnp.exp(logits - m), 0.0)
    probs = ex / jnp.sum(ex, axis=1, keepdims=True)
    p1 = jnp.max(probs, axis=1, keepdims=True)
    e1 = jnp.min(jnp.where(probs == p1, col, _EP), axis=1, keepdims=True)
    probs2 = jnp.where(col == e1, jnp.float32(-1.0), probs)
    p2 = jnp.max(probs2, axis=1, keepdims=True)
    e2 = jnp.min(jnp.where(probs2 == p2, col, _EP), axis=1, keepdims=True)
    d = jnp.exp(p2 - p1)
    w1_ref[...] = 1.0 / (1.0 + d)
    w2_ref[...] = d / (1.0 + d)
    e1_ref[...] = e1
    e2_ref[...] = e2

    oh = (col == e1).astype(jnp.float32) + (col == e2).astype(jnp.float32)
    rr = lax.broadcasted_iota(jnp.int32, (_RCHUNK, _RCHUNK), 0)
    cc = lax.broadcasted_iota(jnp.int32, (_RCHUNK, _RCHUNK), 1)
    tri = (cc < rr).astype(jnp.float32)
    pref = lax.dot_general(tri, oh, (((1,), (0,)), ((), ())),
                           preferred_element_type=jnp.float32)
    base = jnp.where(step == 0, jnp.zeros_like(cts_ref[...]), cts_ref[...])
    run = base + pref
    rank1 = jnp.sum(jnp.where(col == e1, run, 0.0), axis=1, keepdims=True)
    rank2 = jnp.sum(jnp.where(col == e2, run, 0.0), axis=1, keepdims=True)
    r1_ref[...] = rank1.astype(jnp.int32)
    r2_ref[...] = rank2.astype(jnp.int32)
    cts = base + jnp.sum(oh, axis=0, keepdims=True)
    cts_ref[...] = cts

    @pl.when(step == pl.num_programs(0) - 1)
    def _():
        target = jnp.float32(_NTOK * _K / _E)
        ccol = lax.broadcasted_iota(jnp.int32, cts.shape, 1)
        sq = jnp.where(ccol < _E, (cts - target) ** 2, 0.0)
        loss_ref[...] = jnp.sum(sq, axis=1, keepdims=True) / (_E * target * target)


def _router_call(x2d, rwp):
    nsteps = _NTOK // _RCHUNK
    colspec = pl.BlockSpec((_RCHUNK, 1), lambda i: (i, 0))
    return pl.pallas_call(
        _router_body,
        grid=(nsteps,),
        in_specs=[
            pl.BlockSpec((_RCHUNK, _D), lambda i: (i, 0)),
            pl.BlockSpec((_EP, _D), lambda i: (0, 0)),
        ],
        out_specs=[
            colspec, colspec, colspec, colspec, colspec, colspec,
            pl.BlockSpec((1, _EP), lambda i: (0, 0)),
            pl.BlockSpec((1, 1), lambda i: (0, 0)),
        ],
        out_shape=[
            jax.ShapeDtypeStruct((_NTOK, 1), jnp.int32),
            jax.ShapeDtypeStruct((_NTOK, 1), jnp.int32),
            jax.ShapeDtypeStruct((_NTOK, 1), jnp.float32),
            jax.ShapeDtypeStruct((_NTOK, 1), jnp.float32),
            jax.ShapeDtypeStruct((_NTOK, 1), jnp.int32),
            jax.ShapeDtypeStruct((_NTOK, 1), jnp.int32),
            jax.ShapeDtypeStruct((1, _EP), jnp.float32),
            jax.ShapeDtypeStruct((1, 1), jnp.float32),
        ],
    )(x2d, rwp)


def _mlp_body(te_ref, nu_ref, xs_ref, gw_ref, uw_ref, dw_ref, out_ref):
    m = pl.program_id(0)

    @pl.when(m < nu_ref[0])
    def _():
        xs = xs_ref[...]
        g = lax.dot_general(xs, gw_ref[0], (((1,), (1,)), ((), ())),
                            preferred_element_type=jnp.float32)
        u = lax.dot_general(xs, uw_ref[0], (((1,), (1,)), ((), ())),
                            preferred_element_type=jnp.float32)
        h = (g / (1.0 + jnp.exp(-g))) * u
        o = lax.dot_general(h, dw_ref[0], (((1,), (1,)), ((), ())),
                            preferred_element_type=jnp.float32)
        out_ref[...] = o


def _mlp_call(tile_e, nu, xs, gate_W, up_W, down_W):
    grid_spec = pltpu.PrefetchScalarGridSpec(
        num_scalar_prefetch=2,
        grid=(_NT,),
        in_specs=[
            pl.BlockSpec((_TM, _D), lambda m, te, nu: (m, 0)),
            pl.BlockSpec((1, _DFF, _D), lambda m, te, nu: (te[m], 0, 0)),
            pl.BlockSpec((1, _DFF, _D), lambda m, te, nu: (te[m], 0, 0)),
            pl.BlockSpec((1, _D, _DFF), lambda m, te, nu: (te[m], 0, 0)),
        ],
        out_specs=pl.BlockSpec((_TM, _D), lambda m, te, nu: (m, 0)),
    )
    return pl.pallas_call(
        _mlp_body,
        grid_spec=grid_spec,
        out_shape=jax.ShapeDtypeStruct((_MPAD, _D), jnp.float32),
        compiler_params=pltpu.CompilerParams(vmem_limit_bytes=110 * 1024 * 1024),
    )(tile_e, nu, xs, gate_W, up_W, down_W)


def _sc_scatter_call(x2d, e2d, r2d, cnt16):
    ppw = _NPAIR // _NW
    ch = 32
    nch = ppw // ch
    mesh = plsc.VectorSubcoreMesh(core_axis_name="c", subcore_axis_name="s")

    @functools.partial(
        pl.kernel,
        out_type=[
            jax.ShapeDtypeStruct((_MPAD, _D), jnp.float32),
            jax.ShapeDtypeStruct((_NW, nch, ch), jnp.int32),
        ],
        mesh=mesh,
        compiler_params=pltpu.CompilerParams(needs_layout_passes=False),
        scratch_types=[
            pltpu.VMEM((1, ppw), jnp.int32),
            pltpu.VMEM((1, ppw), jnp.int32),
            pltpu.VMEM((16,), jnp.int32),
            pltpu.VMEM((nch, ch), jnp.int32),
            pltpu.VMEM((2, ch, _D), jnp.float32),
            pltpu.SemaphoreType.DMA,
            pltpu.SemaphoreType.DMA,
        ],
    )
    def k(x_hbm, e_hbm, r_hbm, c_hbm, out_hbm, pos_hbm, e_v, r_v, off_v, pos_v,
          buf_v, sem0, sem1):
        wid = lax.axis_index("s") * 2 + lax.axis_index("c")
        tbase = (wid % (_NW // _K)) * ppw
        pltpu.sync_copy(e_hbm.at[pl.ds(wid, 1)], e_v)
        pltpu.sync_copy(r_hbm.at[pl.ds(wid, 1)], r_v)
        pltpu.sync_copy(c_hbm, off_v)
        cnt = off_v[...]
        tiles = jnp.right_shift(cnt + (_TM - 1), _TM.bit_length() - 1)
        off_v[...] = (plsc.cumsum(tiles) - tiles) * _TM
        for j in range(ppw // 16):
            ev = e_v[0, pl.ds(j * 16, 16)]
            rv = r_v[0, pl.ds(j * 16, 16)]
            pv = plsc.load_gather(off_v, [ev]) + rv
            pos_v[j // (ch // 16), pl.ds((j % (ch // 16)) * 16, 16)] = pv
        pltpu.sync_copy(pos_v, pos_hbm.at[wid])
        sems = (sem0, sem1)
        pltpu.sync_copy(x_hbm.at[pl.ds(tbase, ch)], buf_v.at[0])
        cps = [None, None]
        for c in range(nch):
            s = c % 2
            cps[s] = pltpu.async_copy(buf_v.at[s], out_hbm.at[pos_v.at[c]], sems[s])
            if c + 1 < nch:
                if cps[1 - s] is not None:
                    cps[1 - s].wait()
                pltpu.sync_copy(x_hbm.at[pl.ds(tbase + (c + 1) * ch, ch)],
                                buf_v.at[1 - s])
        cps[(nch - 1) % 2].wait()

    return k(x2d, e2d, r2d, cnt16)


def _sc_gather_call(src, idx3d):
    rw = _NPAIR // _NW
    ch = 32
    nch = rw // ch
    nb = 3
    mesh = plsc.VectorSubcoreMesh(core_axis_name="c", subcore_axis_name="s")

    @functools.partial(
        pl.kernel,
        out_type=jax.ShapeDtypeStruct((_NPAIR, _D), jnp.float32),
        mesh=mesh,
        scratch_types=[
            pltpu.VMEM((rw,), jnp.int32),
            pltpu.VMEM((nb, ch, _D), jnp.float32),
            [pltpu.SemaphoreType.DMA] * nb,
            [pltpu.SemaphoreType.DMA] * nb,
        ],
    )
    def k(src_hbm, idx_hbm, out_hbm, idx_v, buf_v, gsems, wsems):
        wid = lax.axis_index("s") * 2 + lax.axis_index("c")
        base = wid * rw
        pltpu.sync_copy(idx_hbm.at[wid], idx_v)
        gcps = [None] * nb
        wcps = [None] * nb
        for c in range(min(nb, nch)):
            gcps[c] = pltpu.async_copy(
                src_hbm.at[idx_v.at[pl.ds(c * ch, ch)]], buf_v.at[c], gsems[c])
        for c in range(nch):
            s = c % nb
            gcps[s].wait()
            wcps[s] = pltpu.async_copy(
                buf_v.at[s], out_hbm.at[pl.ds(base + c * ch, ch)], wsems[s])
            nxt = c + nb
            if nxt < nch:
                wcps[s].wait()
                gcps[s] = pltpu.async_copy(
                    src_hbm.at[idx_v.at[pl.ds(nxt * ch, ch)]], buf_v.at[s], gsems[s])
        for s in range(min(nb, nch)):
            if wcps[s] is not None:
                wcps[s].wait()

    return k(src, idx3d)


def _combine_body(a_ref, b_ref, w1_ref, w2_ref, o_ref):
    o_ref[...] = a_ref[...] * w1_ref[...] + b_ref[...] * w2_ref[...]


def _combine_call(pair_rows, w1, w2):
    nb = _NTOK // _RCHUNK
    return pl.pallas_call(
        _combine_body,
        grid=(nb,),
        in_specs=[
            pl.BlockSpec((_RCHUNK, _D), lambda i: (i, 0)),
            pl.BlockSpec((_RCHUNK, _D), lambda i: (i + nb, 0)),
            pl.BlockSpec((_RCHUNK, 1), lambda i: (i, 0)),
            pl.BlockSpec((_RCHUNK, 1), lambda i: (i, 0)),
        ],
        out_specs=pl.BlockSpec((_RCHUNK, _D), lambda i: (i, 0)),
        out_shape=jax.ShapeDtypeStruct((_NTOK, _D), jnp.float32),
    )(pair_rows, pair_rows, w1, w2)


def kernel(x, router_W, gate_W, up_W, down_W):
    x2d = x.reshape(_NTOK, _D)
    rwp = jnp.zeros((_EP, _D), jnp.float32).at[:_E].set(router_W)

    e1, e2, w1, w2, r1, r2, cts, loss = _router_call(x2d, rwp)

    cnt16 = cts[0].astype(jnp.int32)
    tiles_e = (cnt16[:_E] + _TM - 1) // _TM
    cum_tiles = jnp.cumsum(tiles_e)
    nu = cum_tiles[-1:]
    tile_e = jnp.minimum(
        jnp.searchsorted(cum_tiles, jnp.arange(_NT, dtype=jnp.int32), side="right"),
        _E - 1,
    ).astype(jnp.int32)

    ppw = _NPAIR // _NW
    e2d = jnp.concatenate([e1[:, 0], e2[:, 0]]).reshape(_NW, ppw)
    r2d = jnp.concatenate([r1[:, 0], r2[:, 0]]).reshape(_NW, ppw)

    xs, pos3 = _sc_scatter_call(x2d, e2d, r2d, cnt16)
    outs = _mlp_call(tile_e, nu, xs, gate_W, up_W, down_W)
    pair_rows = _sc_gather_call(outs, pos3.reshape(_NW, ppw))
    final = _combine_call(pair_rows, w1, w2)
    return final.reshape(x.shape), loss[0, 0]


# speedup vs baseline: 2.0740x; 1.0195x over previous
import functools

import jax
import jax.numpy as jnp
from jax import lax
from jax.experimental import pallas as pl
from jax.experimental.pallas import tpu as pltpu
from jax.experimental.pallas import tpu_sc as plsc

_E = 9
_EP = 16
_K = 2
_D = 1024
_DFF = 2048
_NTOK = 4096
_NPAIR = _NTOK * _K
_TM = 512
_NT = 25
_MPAD = _TM * _NT
_NW = 32
_RCHUNK = 1024


def _router_body(x_ref, w_ref, e1_ref, e2_ref, w1_ref, w2_ref, r1_ref, r2_ref,
                 cts_ref, loss_ref, nu_ref, te_ref):
    step = pl.program_id(0)
    x = x_ref[...]
    logits = lax.dot_general(x, w_ref[...], (((1,), (1,)), ((), ())),
                             preferred_element_type=jnp.float32)
    col = lax.broadcasted_iota(jnp.int32, logits.shape, 1)
    valid = col < _E
    logits = jnp.where(valid, logits, jnp.float32(-1e30))
    m = jnp.max(logits, axis=1, keepdims=True)
    ex = jnp.where(valid, jnp.exp(logits - m), 0.0)
    probs = ex / jnp.sum(ex, axis=1, keepdims=True)
    p1 = jnp.max(probs, axis=1, keepdims=True)
    e1 = jnp.min(jnp.where(probs == p1, col, _EP), axis=1, keepdims=True)
    probs2 = jnp.where(col == e1, jnp.float32(-1.0), probs)
    p2 = jnp.max(probs2, axis=1, keepdims=True)
    e2 = jnp.min(jnp.where(probs2 == p2, col, _EP), axis=1, keepdims=True)
    d = jnp.exp(p2 - p1)
    w1_ref[...] = 1.0 / (1.0 + d)
    w2_ref[...] = d / (1.0 + d)
    e1_ref[...] = e1
    e2_ref[...] = e2

    oh = (col == e1).astype(jnp.float32) + (col == e2).astype(jnp.float32)
    rr = lax.broadcasted_iota(jnp.int32, (_RCHUNK, _RCHUNK), 0)
    cc = lax.broadcasted_iota(jnp.int32, (_RCHUNK, _RCHUNK), 1)
    tri = (cc < rr).astype(jnp.float32)
    pref = lax.dot_general(tri, oh, (((1,), (0,)), ((), ())),
                           preferred_element_type=jnp.float32)
    base = jnp.where(step == 0, jnp.zeros_like(cts_ref[...]), cts_ref[...])
    run = base + pref
    rank1 = jnp.sum(jnp.where(col == e1, run, 0.0), axis=1, keepdims=True)
    rank2 = jnp.sum(jnp.where(col == e2, run, 0.0), axis=1, keepdims=True)
    r1_ref[...] = rank1.astype(jnp.int32)
    r2_ref[...] = rank2.astype(jnp.int32)
    cts = base + jnp.sum(oh, axis=0, keepdims=True)
    cts_ref[...] = cts

    @pl.when(step == pl.num_programs(0) - 1)
    def _():
        target = jnp.float32(_NTOK * _K / _E)
        ccol = lax.broadcasted_iota(jnp.int32, cts.shape, 1)
        sq = jnp.where(ccol < _E, (cts - target) ** 2, 0.0)
        loss_ref[...] = jnp.sum(sq, axis=1, keepdims=True) / (_E * target * target)
        tiles = jnp.floor((cts + (_TM - 1)) * (1.0 / _TM))
        icol = lax.broadcasted_iota(jnp.int32, (_EP, _EP), 0)
        jcol = lax.broadcasted_iota(jnp.int32, (_EP, _EP), 1)
        inc_tri = (icol <= jcol).astype(jnp.float32)
        cum = lax.dot_general(tiles, inc_tri, (((1,), (0,)), ((), ())),
                              preferred_element_type=jnp.float32)
        nu_ref[...] = jnp.sum(jnp.where(ccol == _E - 1, cum, 0.0), axis=1,
                              keepdims=True).astype(jnp.int32)
        ti = lax.broadcasted_iota(jnp.int32, (1, 64), 1).astype(jnp.float32)
        acc = jnp.zeros((1, 64), jnp.int32)
        for e in range(_E):
            cum_e = jnp.sum(jnp.where(ccol == e, cum, 0.0), axis=1, keepdims=True)
            acc = acc + (cum_e <= ti).astype(jnp.int32)
        te_ref[...] = jnp.minimum(acc, _E - 1)


def _router_call(x2d, rwp):
    nsteps = _NTOK // _RCHUNK
    colspec = pl.BlockSpec((_RCHUNK, 1), lambda i: (i, 0))
    return pl.pallas_call(
        _router_body,
        grid=(nsteps,),
        in_specs=[
            pl.BlockSpec((_RCHUNK, _D), lambda i: (i, 0)),
            pl.BlockSpec((_EP, _D), lambda i: (0, 0)),
        ],
        out_specs=[
            colspec, colspec, colspec, colspec, colspec, colspec,
            pl.BlockSpec((1, _EP), lambda i: (0, 0)),
            pl.BlockSpec((1, 1), lambda i: (0, 0)),
            pl.BlockSpec((1, 1), lambda i: (0, 0)),
            pl.BlockSpec((1, 64), lambda i: (0, 0)),
        ],
        out_shape=[
            jax.ShapeDtypeStruct((_NTOK, 1), jnp.int32),
            jax.ShapeDtypeStruct((_NTOK, 1), jnp.int32),
            jax.ShapeDtypeStruct((_NTOK, 1), jnp.float32),
            jax.ShapeDtypeStruct((_NTOK, 1), jnp.float32),
            jax.ShapeDtypeStruct((_NTOK, 1), jnp.int32),
            jax.ShapeDtypeStruct((_NTOK, 1), jnp.int32),
            jax.ShapeDtypeStruct((1, _EP), jnp.float32),
            jax.ShapeDtypeStruct((1, 1), jnp.float32),
            jax.ShapeDtypeStruct((1, 1), jnp.int32),
            jax.ShapeDtypeStruct((1, 64), jnp.int32),
        ],
    )(x2d, rwp)


def _mlp_body(te_ref, nu_ref, xs_ref, gw_ref, uw_ref, dw_ref, out_ref):
    m = pl.program_id(0)

    @pl.when(m < nu_ref[0, 0])
    def _():
        xs = xs_ref[...]
        g = lax.dot_general(xs, gw_ref[0], (((1,), (1,)), ((), ())),
                            preferred_element_type=jnp.float32)
        u = lax.dot_general(xs, uw_ref[0], (((1,), (1,)), ((), ())),
                            preferred_element_type=jnp.float32)
        h = (g / (1.0 + jnp.exp(-g))) * u
        o = lax.dot_general(h, dw_ref[0], (((1,), (1,)), ((), ())),
                            preferred_element_type=jnp.float32)
        out_ref[...] = o


def _mlp_call(tile_e, nu, xs, gate_W, up_W, down_W):
    grid_spec = pltpu.PrefetchScalarGridSpec(
        num_scalar_prefetch=2,
        grid=(_NT,),
        in_specs=[
            pl.BlockSpec((_TM, _D), lambda m, te, nu: (m, 0)),
            pl.BlockSpec((1, _DFF, _D), lambda m, te, nu: (te[0, m], 0, 0)),
            pl.BlockSpec((1, _DFF, _D), lambda m, te, nu: (te[0, m], 0, 0)),
            pl.BlockSpec((1, _D, _DFF), lambda m, te, nu: (te[0, m], 0, 0)),
        ],
        out_specs=pl.BlockSpec((_TM, _D), lambda m, te, nu: (m, 0)),
    )
    return pl.pallas_call(
        _mlp_body,
        grid_spec=grid_spec,
        out_shape=jax.ShapeDtypeStruct((_MPAD, _D), jnp.float32),
        compiler_params=pltpu.CompilerParams(vmem_limit_bytes=110 * 1024 * 1024),
    )(tile_e, nu, xs, gate_W, up_W, down_W)


def _sc_scatter_call(x2d, e3d, r3d, cnt16):
    tpw = _NTOK // _NW
    ch = 32
    nch = tpw // ch
    mesh = plsc.VectorSubcoreMesh(core_axis_name="c", subcore_axis_name="s")

    @functools.partial(
        pl.kernel,
        out_type=[
            jax.ShapeDtypeStruct((_MPAD, _D), jnp.float32),
            jax.ShapeDtypeStruct((_K * _NW, nch, ch), jnp.int32),
        ],
        mesh=mesh,
        compiler_params=pltpu.CompilerParams(needs_layout_passes=False),
        scratch_types=[
            pltpu.VMEM((_K, tpw), jnp.int32),
            pltpu.VMEM((_K, tpw), jnp.int32),
            pltpu.VMEM((16,), jnp.int32),
            pltpu.VMEM((nch, ch), jnp.int32),
            pltpu.VMEM((nch, ch), jnp.int32),
            pltpu.VMEM((2, ch, _D), jnp.float32),
            pltpu.SemaphoreType.DMA,
            pltpu.SemaphoreType.DMA,
        ],
    )
    def k(x_hbm, e_hbm, r_hbm, c_hbm, out_hbm, pos_hbm, e_v, r_v, off_v,
          p1_v, p2_v, buf_v, sem0, sem1):
        wid = lax.axis_index("s") * 2 + lax.axis_index("c")
        tbase = wid * tpw
        pltpu.sync_copy(e_hbm.at[wid], e_v)
        pltpu.sync_copy(r_hbm.at[wid], r_v)
        pltpu.sync_copy(c_hbm, off_v)
        cnt = off_v[...]
        tiles = jnp.right_shift(cnt + (_TM - 1), _TM.bit_length() - 1)
        off_v[...] = (plsc.cumsum(tiles) - tiles) * _TM
        nvch = ch // 16
        for j in range(tpw // 16):
            for kk, p_v in ((0, p1_v), (1, p2_v)):
                ev = e_v[kk, pl.ds(j * 16, 16)]
                rv = r_v[kk, pl.ds(j * 16, 16)]
                pv = plsc.load_gather(off_v, [ev]) + rv
                p_v[j // nvch, pl.ds((j % nvch) * 16, 16)] = pv
        pltpu.sync_copy(p1_v, pos_hbm.at[wid])
        pltpu.sync_copy(p2_v, pos_hbm.at[_NW + wid])
        sems = (sem0, sem1)
        pltpu.sync_copy(x_hbm.at[pl.ds(tbase, ch)], buf_v.at[0])
        cps = [None, None]
        for c in range(nch):
            s = c % 2
            cpa = pltpu.async_copy(buf_v.at[s], out_hbm.at[p1_v.at[c]], sems[s])
            cpb = pltpu.async_copy(buf_v.at[s], out_hbm.at[p2_v.at[c]], sems[s])
            cps[s] = (cpa, cpb)
            if c + 1 < nch:
                if cps[1 - s] is not None:
                    cps[1 - s][0].wait()
                    cps[1 - s][1].wait()
                pltpu.sync_copy(x_hbm.at[pl.ds(tbase + (c + 1) * ch, ch)],
                                buf_v.at[1 - s])
        cps[(nch - 1) % 2][0].wait()
        cps[(nch - 1) % 2][1].wait()

    return k(x2d, e3d, r3d, cnt16)


def _sc_gather_call(src, idx3d):
    rw = _NPAIR // _NW
    ch = 32
    nch = rw // ch
    nb = 3
    mesh = plsc.VectorSubcoreMesh(core_axis_name="c", subcore_axis_name="s")

    @functools.partial(
        pl.kernel,
        out_type=jax.ShapeDtypeStruct((_NPAIR, _D), jnp.float32),
        mesh=mesh,
        scratch_types=[
            pltpu.VMEM((rw,), jnp.int32),
            pltpu.VMEM((nb, ch, _D), jnp.float32),
            [pltpu.SemaphoreType.DMA] * nb,
            [pltpu.SemaphoreType.DMA] * nb,
        ],
    )
    def k(src_hbm, idx_hbm, out_hbm, idx_v, buf_v, gsems, wsems):
        wid = lax.axis_index("s") * 2 + lax.axis_index("c")
        base = wid * rw
        pltpu.sync_copy(idx_hbm.at[wid], idx_v)
        gcps = [None] * nb
        wcps = [None] * nb
        for c in range(min(nb, nch)):
            gcps[c] = pltpu.async_copy(
                src_hbm.at[idx_v.at[pl.ds(c * ch, ch)]], buf_v.at[c], gsems[c])
        for c in range(nch):
            s = c % nb
            gcps[s].wait()
            wcps[s] = pltpu.async_copy(
                buf_v.at[s], out_hbm.at[pl.ds(base + c * ch, ch)], wsems[s])
            nxt = c + nb
            if nxt < nch:
                wcps[s].wait()
                gcps[s] = pltpu.async_copy(
                    src_hbm.at[idx_v.at[pl.ds(nxt * ch, ch)]], buf_v.at[s], gsems[s])
        for s in range(min(nb, nch)):
            if wcps[s] is not None:
                wcps[s].wait()

    return k(src, idx3d)


def _combine_body(a_ref, b_ref, w1_ref, w2_ref, o_ref):
    o_ref[...] = a_ref[...] * w1_ref[...] + b_ref[...] * w2_ref[...]


def _combine_call(pair_rows, w1, w2):
    nb = _NTOK // _RCHUNK
    return pl.pallas_call(
        _combine_body,
        grid=(nb,),
        in_specs=[
            pl.BlockSpec((_RCHUNK, _D), lambda i: (i, 0)),
            pl.BlockSpec((_RCHUNK, _D), lambda i: (i + nb, 0)),
            pl.BlockSpec((_RCHUNK, 1), lambda i: (i, 0)),
            pl.BlockSpec((_RCHUNK, 1), lambda i: (i, 0)),
        ],
        out_specs=pl.BlockSpec((_RCHUNK, _D), lambda i: (i, 0)),
        out_shape=jax.ShapeDtypeStruct((_NTOK, _D), jnp.float32),
    )(pair_rows, pair_rows, w1, w2)


def kernel(x, router_W, gate_W, up_W, down_W):
    x2d = x.reshape(_NTOK, _D)
    rwp = jnp.zeros((_EP, _D), jnp.float32).at[:_E].set(router_W)

    e1, e2, w1, w2, r1, r2, cts, loss, nu, tile_e = _router_call(x2d, rwp)

    cnt16 = cts[0].astype(jnp.int32)
    tpw = _NTOK // _NW
    e3d = jnp.concatenate(
        [e1.reshape(_NW, 1, tpw), e2.reshape(_NW, 1, tpw)], axis=1)
    r3d = jnp.concatenate(
        [r1.reshape(_NW, 1, tpw), r2.reshape(_NW, 1, tpw)], axis=1)

    xs, pos3 = _sc_scatter_call(x2d, e3d, r3d, cnt16)
    outs = _mlp_call(tile_e, nu, xs, gate_W, up_W, down_W)
    pair_rows = _sc_gather_call(outs, pos3.reshape(_NW, _NPAIR // _NW))
    final = _combine_call(pair_rows, w1, w2)
    return final.reshape(x.shape), loss[0, 0]


# speedup vs baseline: 2.2302x; 1.0753x over previous
import functools

import jax
import jax.numpy as jnp
from jax import lax
from jax.experimental import pallas as pl
from jax.experimental.pallas import tpu as pltpu
from jax.experimental.pallas import tpu_sc as plsc

_E = 9
_EP = 16
_K = 2
_D = 1024
_DFF = 2048
_NTOK = 4096
_NPAIR = _NTOK * _K
_TM = 512
_NT = 25
_MPAD = _TM * _NT
_NW = 32
_RCHUNK = 1024


def _router_body(x_ref, w_ref, e1_ref, e2_ref, w1_ref, w2_ref, r1_ref, r2_ref,
                 cts_ref, loss_ref, nu_ref, te_ref):
    step = pl.program_id(0)
    x = x_ref[...]
    logits = lax.dot_general(x, w_ref[...], (((1,), (1,)), ((), ())),
                             preferred_element_type=jnp.float32)
    col = lax.broadcasted_iota(jnp.int32, logits.shape, 1)
    valid = col < _E
    logits = jnp.where(valid, logits, jnp.float32(-1e30))
    m = jnp.max(logits, axis=1, keepdims=True)
    ex = jnp.where(valid, jnp.exp(logits - m), 0.0)
    probs = ex / jnp.sum(ex, axis=1, keepdims=True)
    p1 = jnp.max(probs, axis=1, keepdims=True)
    e1 = jnp.min(jnp.where(probs == p1, col, _EP), axis=1, keepdims=True)
    probs2 = jnp.where(col == e1, jnp.float32(-1.0), probs)
    p2 = jnp.max(probs2, axis=1, keepdims=True)
    e2 = jnp.min(jnp.where(probs2 == p2, col, _EP), axis=1, keepdims=True)
    d = jnp.exp(p2 - p1)
    w1_ref[...] = 1.0 / (1.0 + d)
    w2_ref[...] = d / (1.0 + d)
    e1_ref[...] = e1
    e2_ref[...] = e2

    oh = (col == e1).astype(jnp.float32) + (col == e2).astype(jnp.float32)
    rr = lax.broadcasted_iota(jnp.int32, (_RCHUNK, _RCHUNK), 0)
    cc = lax.broadcasted_iota(jnp.int32, (_RCHUNK, _RCHUNK), 1)
    tri = (cc < rr).astype(jnp.float32)
    pref = lax.dot_general(tri, oh, (((1,), (0,)), ((), ())),
                           preferred_element_type=jnp.float32)
    base = jnp.where(step == 0, jnp.zeros_like(cts_ref[...]), cts_ref[...])
    run = base + pref
    rank1 = jnp.sum(jnp.where(col == e1, run, 0.0), axis=1, keepdims=True)
    rank2 = jnp.sum(jnp.where(col == e2, run, 0.0), axis=1, keepdims=True)
    r1_ref[...] = rank1.astype(jnp.int32)
    r2_ref[...] = rank2.astype(jnp.int32)
    cts = base + jnp.sum(oh, axis=0, keepdims=True)
    cts_ref[...] = cts

    @pl.when(step == pl.num_programs(0) - 1)
    def _():
        target = jnp.float32(_NTOK * _K / _E)
        ccol = lax.broadcasted_iota(jnp.int32, cts.shape, 1)
        sq = jnp.where(ccol < _E, (cts - target) ** 2, 0.0)
        loss_ref[...] = jnp.sum(sq, axis=1, keepdims=True) / (_E * target * target)
        tiles = jnp.floor((cts + (_TM - 1)) * (1.0 / _TM))
        icol = lax.broadcasted_iota(jnp.int32, (_EP, _EP), 0)
        jcol = lax.broadcasted_iota(jnp.int32, (_EP, _EP), 1)
        inc_tri = (icol <= jcol).astype(jnp.float32)
        cum = lax.dot_general(tiles, inc_tri, (((1,), (0,)), ((), ())),
                              preferred_element_type=jnp.float32)
        nu_ref[...] = jnp.sum(jnp.where(ccol == _E - 1, cum, 0.0), axis=1,
                              keepdims=True).astype(jnp.int32)
        ti = lax.broadcasted_iota(jnp.int32, (1, 64), 1).astype(jnp.float32)
        acc = jnp.zeros((1, 64), jnp.int32)
        for e in range(_E):
            cum_e = jnp.sum(jnp.where(ccol == e, cum, 0.0), axis=1, keepdims=True)
            acc = acc + (cum_e <= ti).astype(jnp.int32)
        te_ref[...] = jnp.minimum(acc, _E - 1)


def _router_call(x2d, rwp):
    nsteps = _NTOK // _RCHUNK
    colspec = pl.BlockSpec((_RCHUNK, 1), lambda i: (i, 0))
    return pl.pallas_call(
        _router_body,
        grid=(nsteps,),
        in_specs=[
            pl.BlockSpec((_RCHUNK, _D), lambda i: (i, 0)),
            pl.BlockSpec((_EP, _D), lambda i: (0, 0)),
        ],
        out_specs=[
            colspec, colspec, colspec, colspec, colspec, colspec,
            pl.BlockSpec((1, _EP), lambda i: (0, 0)),
            pl.BlockSpec((1, 1), lambda i: (0, 0)),
            pl.BlockSpec((1, 1), lambda i: (0, 0)),
            pl.BlockSpec((1, 64), lambda i: (0, 0)),
        ],
        out_shape=[
            jax.ShapeDtypeStruct((_NTOK, 1), jnp.int32),
            jax.ShapeDtypeStruct((_NTOK, 1), jnp.int32),
            jax.ShapeDtypeStruct((_NTOK, 1), jnp.float32),
            jax.ShapeDtypeStruct((_NTOK, 1), jnp.float32),
            jax.ShapeDtypeStruct((_NTOK, 1), jnp.int32),
            jax.ShapeDtypeStruct((_NTOK, 1), jnp.int32),
            jax.ShapeDtypeStruct((1, _EP), jnp.float32),
            jax.ShapeDtypeStruct((1, 1), jnp.float32),
            jax.ShapeDtypeStruct((1, 1), jnp.int32),
            jax.ShapeDtypeStruct((1, 64), jnp.int32),
        ],
    )(x2d, rwp)


def _mlp_body(te_ref, nu_ref, xs_ref, gw_ref, uw_ref, dw_ref, out_ref):
    m = pl.program_id(0)

    @pl.when(m < nu_ref[0, 0])
    def _():
        xs = xs_ref[...]
        g = lax.dot_general(xs, gw_ref[0], (((1,), (1,)), ((), ())),
                            preferred_element_type=jnp.float32)
        u = lax.dot_general(xs, uw_ref[0], (((1,), (1,)), ((), ())),
                            preferred_element_type=jnp.float32)
        h = (g / (1.0 + jnp.exp(-g))) * u
        o = lax.dot_general(h, dw_ref[0], (((1,), (1,)), ((), ())),
                            preferred_element_type=jnp.float32)
        out_ref[...] = o


def _mlp_call(tile_e, nu, xs, gate_W, up_W, down_W):
    grid_spec = pltpu.PrefetchScalarGridSpec(
        num_scalar_prefetch=2,
        grid=(_NT,),
        in_specs=[
            pl.BlockSpec((_TM, _D), lambda m, te, nu: (m, 0)),
            pl.BlockSpec((1, _DFF, _D), lambda m, te, nu: (te[0, m], 0, 0)),
            pl.BlockSpec((1, _DFF, _D), lambda m, te, nu: (te[0, m], 0, 0)),
            pl.BlockSpec((1, _D, _DFF), lambda m, te, nu: (te[0, m], 0, 0)),
        ],
        out_specs=pl.BlockSpec((_TM, _D), lambda m, te, nu: (m, 0)),
    )
    return pl.pallas_call(
        _mlp_body,
        grid_spec=grid_spec,
        out_shape=jax.ShapeDtypeStruct((_MPAD, _D), jnp.float32),
        compiler_params=pltpu.CompilerParams(vmem_limit_bytes=110 * 1024 * 1024),
    )(tile_e, nu, xs, gate_W, up_W, down_W)


def _sc_scatter_call(x2d, e3d, r3d, cnt16):
    tpw = _NTOK // _NW
    ch = 32
    nch = tpw // ch
    mesh = plsc.VectorSubcoreMesh(core_axis_name="c", subcore_axis_name="s")

    @functools.partial(
        pl.kernel,
        out_type=[
            jax.ShapeDtypeStruct((_MPAD, _D), jnp.float32),
            jax.ShapeDtypeStruct((_K * _NW, nch, ch), jnp.int32),
        ],
        mesh=mesh,
        compiler_params=pltpu.CompilerParams(needs_layout_passes=False),
        scratch_types=[
            pltpu.VMEM((_K, tpw), jnp.int32),
            pltpu.VMEM((_K, tpw), jnp.int32),
            pltpu.VMEM((16,), jnp.int32),
            pltpu.VMEM((nch, ch), jnp.int32),
            pltpu.VMEM((nch, ch), jnp.int32),
            pltpu.VMEM((2, ch, _D), jnp.float32),
            pltpu.SemaphoreType.DMA,
            pltpu.SemaphoreType.DMA,
        ],
    )
    def k(x_hbm, e_hbm, r_hbm, c_hbm, out_hbm, pos_hbm, e_v, r_v, off_v,
          p1_v, p2_v, buf_v, sem0, sem1):
        wid = lax.axis_index("s") * 2 + lax.axis_index("c")
        tbase = wid * tpw
        pltpu.sync_copy(e_hbm.at[wid], e_v)
        pltpu.sync_copy(r_hbm.at[wid], r_v)
        pltpu.sync_copy(c_hbm, off_v)
        cnt = off_v[...]
        tiles = jnp.right_shift(cnt + (_TM - 1), _TM.bit_length() - 1)
        off_v[...] = (plsc.cumsum(tiles) - tiles) * _TM
        nvch = ch // 16
        for j in range(tpw // 16):
            for kk, p_v in ((0, p1_v), (1, p2_v)):
                ev = e_v[kk, pl.ds(j * 16, 16)]
                rv = r_v[kk, pl.ds(j * 16, 16)]
                pv = plsc.load_gather(off_v, [ev]) + rv
                p_v[j // nvch, pl.ds((j % nvch) * 16, 16)] = pv
        pltpu.sync_copy(p1_v, pos_hbm.at[wid])
        pltpu.sync_copy(p2_v, pos_hbm.at[_NW + wid])
        sems = (sem0, sem1)
        pltpu.sync_copy(x_hbm.at[pl.ds(tbase, ch)], buf_v.at[0])
        cps = [None, None]
        for c in range(nch):
            s = c % 2
            cpa = pltpu.async_copy(buf_v.at[s], out_hbm.at[p1_v.at[c]], sems[s])
            cpb = pltpu.async_copy(buf_v.at[s], out_hbm.at[p2_v.at[c]], sems[s])
            cps[s] = (cpa, cpb)
            if c + 1 < nch:
                if cps[1 - s] is not None:
                    cps[1 - s][0].wait()
                    cps[1 - s][1].wait()
                pltpu.sync_copy(x_hbm.at[pl.ds(tbase + (c + 1) * ch, ch)],
                                buf_v.at[1 - s])
        cps[(nch - 1) % 2][0].wait()
        cps[(nch - 1) % 2][1].wait()

    return k(x2d, e3d, r3d, cnt16)


def _sc_combine_call(outs, pos3, wcat):
    tpw = _NTOK // _NW
    ch = 16
    nch = tpw // ch
    nvr = _D // 16
    mesh = plsc.VectorSubcoreMesh(core_axis_name="c", subcore_axis_name="s")

    @functools.partial(
        pl.kernel,
        out_type=jax.ShapeDtypeStruct((_NTOK, _D), jnp.float32),
        mesh=mesh,
        compiler_params=pltpu.CompilerParams(needs_layout_passes=False),
        scratch_types=[
            pltpu.VMEM((nch // 2, 2 * ch), jnp.int32),
            pltpu.VMEM((nch // 2, 2 * ch), jnp.int32),
            pltpu.VMEM((_K, tpw), jnp.float32),
            pltpu.VMEM((2, ch, _D), jnp.float32),
            pltpu.VMEM((2, ch, _D), jnp.float32),
            pltpu.VMEM((2, ch, _D), jnp.float32),
            [pltpu.SemaphoreType.DMA] * 2,
            [pltpu.SemaphoreType.DMA] * 2,
            [pltpu.SemaphoreType.DMA] * 2,
        ],
    )
    def k(src_hbm, pos_hbm, w_hbm, out_hbm, p1_v, p2_v, w_v, bufa_v, bufb_v,
          bufo_v, asems, bsems, wsems):
        wid = lax.axis_index("s") * 2 + lax.axis_index("c")
        base = wid * tpw
        pltpu.sync_copy(pos_hbm.at[wid], p1_v)
        pltpu.sync_copy(pos_hbm.at[_NW + wid], p2_v)
        pltpu.sync_copy(w_hbm.at[wid], w_v)

        def chunk_idx(p_v, c):
            return p_v[c // 2, pl.ds((c % 2) * ch, ch)]

        def start_gathers(c, s):
            ga = pltpu.async_copy(src_hbm.at[chunk_idx(p1_v, c)],
                                  bufa_v.at[s], asems[s])
            gb = pltpu.async_copy(src_hbm.at[chunk_idx(p2_v, c)],
                                  bufb_v.at[s], bsems[s])
            return ga, gb

        gcps = [start_gathers(0, 0), start_gathers(1, 1)]
        wcps = [None, None]
        for c in range(nch):
            s = c % 2
            gcps[s][0].wait()
            gcps[s][1].wait()
            if wcps[s] is not None:
                wcps[s].wait()

            def row_body(r, carry, c=c, s=s):
                lane0 = jnp.zeros((16,), jnp.int32)
                ridx = lane0 + (c * ch + r)
                wa = plsc.load_gather(w_v, [lane0, ridx])
                wb = plsc.load_gather(w_v, [lane0 + 1, ridx])

                def vec_body(j, carry2):
                    a = bufa_v[s, r, pl.ds(j * 16, 16)]
                    b = bufb_v[s, r, pl.ds(j * 16, 16)]
                    bufo_v[s, r, pl.ds(j * 16, 16)] = wa * a + wb * b
                    return carry2

                lax.fori_loop(0, nvr, vec_body, 0, unroll=4)
                return carry

            lax.fori_loop(0, ch, row_body, 0)
            wcps[s] = pltpu.async_copy(
                bufo_v.at[s], out_hbm.at[pl.ds(base + c * ch, ch)], wsems[s])
            if c + 2 < nch:
                gcps[s] = start_gathers(c + 2, s)
        wcps[0].wait()
        wcps[1].wait()

    return k(outs, pos3, wcat)


def _sc_gather_call(src, idx3d):
    rw = _NPAIR // _NW
    ch = 32
    nch = rw // ch
    nb = 3
    mesh = plsc.VectorSubcoreMesh(core_axis_name="c", subcore_axis_name="s")

    @functools.partial(
        pl.kernel,
        out_type=jax.ShapeDtypeStruct((_NPAIR, _D), jnp.float32),
        mesh=mesh,
        scratch_types=[
            pltpu.VMEM((rw,), jnp.int32),
            pltpu.VMEM((nb, ch, _D), jnp.float32),
            [pltpu.SemaphoreType.DMA] * nb,
            [pltpu.SemaphoreType.DMA] * nb,
        ],
    )
    def k(src_hbm, idx_hbm, out_hbm, idx_v, buf_v, gsems, wsems):
        wid = lax.axis_index("s") * 2 + lax.axis_index("c")
        base = wid * rw
        pltpu.sync_copy(idx_hbm.at[wid], idx_v)
        gcps = [None] * nb
        wcps = [None] * nb
        for c in range(min(nb, nch)):
            gcps[c] = pltpu.async_copy(
                src_hbm.at[idx_v.at[pl.ds(c * ch, ch)]], buf_v.at[c], gsems[c])
        for c in range(nch):
            s = c % nb
            gcps[s].wait()
            wcps[s] = pltpu.async_copy(
                buf_v.at[s], out_hbm.at[pl.ds(base + c * ch, ch)], wsems[s])
            nxt = c + nb
            if nxt < nch:
                wcps[s].wait()
                gcps[s] = pltpu.async_copy(
                    src_hbm.at[idx_v.at[pl.ds(nxt * ch, ch)]], buf_v.at[s], gsems[s])
        for s in range(min(nb, nch)):
            if wcps[s] is not None:
                wcps[s].wait()

    return k(src, idx3d)


def _combine_body(a_ref, b_ref, w1_ref, w2_ref, o_ref):
    o_ref[...] = a_ref[...] * w1_ref[...] + b_ref[...] * w2_ref[...]


def _combine_call(pair_rows, w1, w2):
    nb = _NTOK // _RCHUNK
    return pl.pallas_call(
        _combine_body,
        grid=(nb,),
        in_specs=[
            pl.BlockSpec((_RCHUNK, _D), lambda i: (i, 0)),
            pl.BlockSpec((_RCHUNK, _D), lambda i: (i + nb, 0)),
            pl.BlockSpec((_RCHUNK, 1), lambda i: (i, 0)),
            pl.BlockSpec((_RCHUNK, 1), lambda i: (i, 0)),
        ],
        out_specs=pl.BlockSpec((_RCHUNK, _D), lambda i: (i, 0)),
        out_shape=jax.ShapeDtypeStruct((_NTOK, _D), jnp.float32),
    )(pair_rows, pair_rows, w1, w2)


def kernel(x, router_W, gate_W, up_W, down_W):
    x2d = x.reshape(_NTOK, _D)
    rwp = jnp.zeros((_EP, _D), jnp.float32).at[:_E].set(router_W)

    e1, e2, w1, w2, r1, r2, cts, loss, nu, tile_e = _router_call(x2d, rwp)

    cnt16 = cts[0].astype(jnp.int32)
    tpw = _NTOK // _NW
    e3d = jnp.concatenate(
        [e1.reshape(_NW, 1, tpw), e2.reshape(_NW, 1, tpw)], axis=1)
    r3d = jnp.concatenate(
        [r1.reshape(_NW, 1, tpw), r2.reshape(_NW, 1, tpw)], axis=1)

    wcat = jnp.concatenate(
        [w1.reshape(_NW, 1, tpw), w2.reshape(_NW, 1, tpw)], axis=1)

    xs, pos3 = _sc_scatter_call(x2d, e3d, r3d, cnt16)
    outs = _mlp_call(tile_e, nu, xs, gate_W, up_W, down_W)
    final = _sc_combine_call(outs, pos3, wcat)
    return final.reshape(x.shape), loss[0, 0]


# speedup vs baseline: 2.2321x; 1.0009x over previous
import functools

import jax
import jax.numpy as jnp
from jax import lax
from jax.experimental import pallas as pl
from jax.experimental.pallas import tpu as pltpu
from jax.experimental.pallas import tpu_sc as plsc

_E = 9
_EP = 16
_K = 2
_D = 1024
_DFF = 2048
_NTOK = 4096
_NPAIR = _NTOK * _K
_TM = 512
_NT = 25
_MPAD = _TM * _NT
_NW = 32
_RCHUNK = 512


def _router_body(x_ref, w_ref, e1_ref, e2_ref, w1_ref, w2_ref, r1_ref, r2_ref,
                 cts_ref, loss_ref, nu_ref, te_ref):
    step = pl.program_id(0)
    x = x_ref[...]
    logits = lax.dot_general(x, w_ref[...], (((1,), (1,)), ((), ())),
                             preferred_element_type=jnp.float32)
    col = lax.broadcasted_iota(jnp.int32, logits.shape, 1)
    valid = col < _E
    logits = jnp.where(valid, logits, jnp.float32(-1e30))
    m = jnp.max(logits, axis=1, keepdims=True)
    ex = jnp.where(valid, jnp.exp(logits - m), 0.0)
    probs = ex / jnp.sum(ex, axis=1, keepdims=True)
    p1 = jnp.max(probs, axis=1, keepdims=True)
    e1 = jnp.min(jnp.where(probs == p1, col, _EP), axis=1, keepdims=True)
    probs2 = jnp.where(col == e1, jnp.float32(-1.0), probs)
    p2 = jnp.max(probs2, axis=1, keepdims=True)
    e2 = jnp.min(jnp.where(probs2 == p2, col, _EP), axis=1, keepdims=True)
    d = jnp.exp(p2 - p1)
    w1_ref[...] = 1.0 / (1.0 + d)
    w2_ref[...] = d / (1.0 + d)
    e1_ref[...] = e1
    e2_ref[...] = e2

    oh = (col == e1).astype(jnp.float32) + (col == e2).astype(jnp.float32)
    rr = lax.broadcasted_iota(jnp.int32, (_RCHUNK, _RCHUNK), 0)
    cc = lax.broadcasted_iota(jnp.int32, (_RCHUNK, _RCHUNK), 1)
    tri = (cc < rr).astype(jnp.float32)
    pref = lax.dot_general(tri, oh, (((1,), (0,)), ((), ())),
                           preferred_element_type=jnp.float32)
    base = jnp.where(step == 0, jnp.zeros_like(cts_ref[...]), cts_ref[...])
    run = base + pref
    rank1 = jnp.sum(jnp.where(col == e1, run, 0.0), axis=1, keepdims=True)
    rank2 = jnp.sum(jnp.where(col == e2, run, 0.0), axis=1, keepdims=True)
    r1_ref[...] = rank1.astype(jnp.int32)
    r2_ref[...] = rank2.astype(jnp.int32)
    cts = base + jnp.sum(oh, axis=0, keepdims=True)
    cts_ref[...] = cts

    @pl.when(step == pl.num_programs(0) - 1)
    def _():
        target = jnp.float32(_NTOK * _K / _E)
        ccol = lax.broadcasted_iota(jnp.int32, cts.shape, 1)
        sq = jnp.where(ccol < _E, (cts - target) ** 2, 0.0)
        loss_ref[...] = jnp.sum(sq, axis=1, keepdims=True) / (_E * target * target)
        tiles = jnp.floor((cts + (_TM - 1)) * (1.0 / _TM))
        icol = lax.broadcasted_iota(jnp.int32, (_EP, _EP), 0)
        jcol = lax.broadcasted_iota(jnp.int32, (_EP, _EP), 1)
        inc_tri = (icol <= jcol).astype(jnp.float32)
        cum = lax.dot_general(tiles, inc_tri, (((1,), (0,)), ((), ())),
                              preferred_element_type=jnp.float32)
        nu_ref[...] = jnp.sum(jnp.where(ccol == _E - 1, cum, 0.0), axis=1,
                              keepdims=True).astype(jnp.int32)
        ti = lax.broadcasted_iota(jnp.int32, (1, 64), 1).astype(jnp.float32)
        acc = jnp.zeros((1, 64), jnp.int32)
        for e in range(_E):
            cum_e = jnp.sum(jnp.where(ccol == e, cum, 0.0), axis=1, keepdims=True)
            acc = acc + (cum_e <= ti).astype(jnp.int32)
        te_ref[...] = jnp.minimum(acc, _E - 1)


def _router_call(x2d, rwp):
    nsteps = _NTOK // _RCHUNK
    colspec = pl.BlockSpec((_RCHUNK, 1), lambda i: (i, 0))
    return pl.pallas_call(
        _router_body,
        grid=(nsteps,),
        in_specs=[
            pl.BlockSpec((_RCHUNK, _D), lambda i: (i, 0)),
            pl.BlockSpec((_EP, _D), lambda i: (0, 0)),
        ],
        out_specs=[
            colspec, colspec, colspec, colspec, colspec, colspec,
            pl.BlockSpec((1, _EP), lambda i: (0, 0)),
            pl.BlockSpec((1, 1), lambda i: (0, 0)),
            pl.BlockSpec((1, 1), lambda i: (0, 0)),
            pl.BlockSpec((1, 64), lambda i: (0, 0)),
        ],
        out_shape=[
            jax.ShapeDtypeStruct((_NTOK, 1), jnp.int32),
            jax.ShapeDtypeStruct((_NTOK, 1), jnp.int32),
            jax.ShapeDtypeStruct((_NTOK, 1), jnp.float32),
            jax.ShapeDtypeStruct((_NTOK, 1), jnp.float32),
            jax.ShapeDtypeStruct((_NTOK, 1), jnp.int32),
            jax.ShapeDtypeStruct((_NTOK, 1), jnp.int32),
            jax.ShapeDtypeStruct((1, _EP), jnp.float32),
            jax.ShapeDtypeStruct((1, 1), jnp.float32),
            jax.ShapeDtypeStruct((1, 1), jnp.int32),
            jax.ShapeDtypeStruct((1, 64), jnp.int32),
        ],
    )(x2d, rwp)


def _mlp_body(te_ref, nu_ref, xs_ref, gw_ref, uw_ref, dw_ref, out_ref):
    m = pl.program_id(0)

    @pl.when(m < nu_ref[0, 0])
    def _():
        xs = xs_ref[...]
        g = lax.dot_general(xs, gw_ref[0], (((1,), (1,)), ((), ())),
                            preferred_element_type=jnp.float32)
        u = lax.dot_general(xs, uw_ref[0], (((1,), (1,)), ((), ())),
                            preferred_element_type=jnp.float32)
        h = (g / (1.0 + jnp.exp(-g))) * u
        o = lax.dot_general(h, dw_ref[0], (((1,), (1,)), ((), ())),
                            preferred_element_type=jnp.float32)
        out_ref[...] = o


def _mlp_call(tile_e, nu, xs, gate_W, up_W, down_W):
    grid_spec = pltpu.PrefetchScalarGridSpec(
        num_scalar_prefetch=2,
        grid=(_NT,),
        in_specs=[
            pl.BlockSpec((_TM, _D), lambda m, te, nu: (m, 0)),
            pl.BlockSpec((1, _DFF, _D), lambda m, te, nu: (te[0, m], 0, 0)),
            pl.BlockSpec((1, _DFF, _D), lambda m, te, nu: (te[0, m], 0, 0)),
            pl.BlockSpec((1, _D, _DFF), lambda m, te, nu: (te[0, m], 0, 0)),
        ],
        out_specs=pl.BlockSpec((_TM, _D), lambda m, te, nu: (m, 0)),
    )
    return pl.pallas_call(
        _mlp_body,
        grid_spec=grid_spec,
        out_shape=jax.ShapeDtypeStruct((_MPAD, _D), jnp.float32),
        compiler_params=pltpu.CompilerParams(vmem_limit_bytes=110 * 1024 * 1024),
    )(tile_e, nu, xs, gate_W, up_W, down_W)


def _sc_scatter_call(x2d, e3d, r3d, cnt16):
    tpw = _NTOK // _NW
    ch = 32
    nch = tpw // ch
    mesh = plsc.VectorSubcoreMesh(core_axis_name="c", subcore_axis_name="s")

    @functools.partial(
        pl.kernel,
        out_type=[
            jax.ShapeDtypeStruct((_MPAD, _D), jnp.float32),
            jax.ShapeDtypeStruct((_K * _NW, nch, ch), jnp.int32),
        ],
        mesh=mesh,
        compiler_params=pltpu.CompilerParams(needs_layout_passes=False),
        scratch_types=[
            pltpu.VMEM((_K, tpw), jnp.int32),
            pltpu.VMEM((_K, tpw), jnp.int32),
            pltpu.VMEM((16,), jnp.int32),
            pltpu.VMEM((nch, ch), jnp.int32),
            pltpu.VMEM((nch, ch), jnp.int32),
            pltpu.VMEM((2, ch, _D), jnp.float32),
            pltpu.SemaphoreType.DMA,
            pltpu.SemaphoreType.DMA,
        ],
    )
    def k(x_hbm, e_hbm, r_hbm, c_hbm, out_hbm, pos_hbm, e_v, r_v, off_v,
          p1_v, p2_v, buf_v, sem0, sem1):
        wid = lax.axis_index("s") * 2 + lax.axis_index("c")
        tbase = wid * tpw
        pltpu.sync_copy(e_hbm.at[wid], e_v)
        pltpu.sync_copy(r_hbm.at[wid], r_v)
        pltpu.sync_copy(c_hbm, off_v)
        cnt = off_v[...]
        tiles = jnp.right_shift(cnt + (_TM - 1), _TM.bit_length() - 1)
        off_v[...] = (plsc.cumsum(tiles) - tiles) * _TM
        nvch = ch // 16
        for j in range(tpw // 16):
            for kk, p_v in ((0, p1_v), (1, p2_v)):
                ev = e_v[kk, pl.ds(j * 16, 16)]
                rv = r_v[kk, pl.ds(j * 16, 16)]
                pv = plsc.load_gather(off_v, [ev]) + rv
                p_v[j // nvch, pl.ds((j % nvch) * 16, 16)] = pv
        pltpu.sync_copy(p1_v, pos_hbm.at[wid])
        pltpu.sync_copy(p2_v, pos_hbm.at[_NW + wid])
        sems = (sem0, sem1)
        pltpu.sync_copy(x_hbm.at[pl.ds(tbase, ch)], buf_v.at[0])
        cps = [None, None]
        for c in range(nch):
            s = c % 2
            cpa = pltpu.async_copy(buf_v.at[s], out_hbm.at[p1_v.at[c]], sems[s])
            cpb = pltpu.async_copy(buf_v.at[s], out_hbm.at[p2_v.at[c]], sems[s])
            cps[s] = (cpa, cpb)
            if c + 1 < nch:
                if cps[1 - s] is not None:
                    cps[1 - s][0].wait()
                    cps[1 - s][1].wait()
                pltpu.sync_copy(x_hbm.at[pl.ds(tbase + (c + 1) * ch, ch)],
                                buf_v.at[1 - s])
        cps[(nch - 1) % 2][0].wait()
        cps[(nch - 1) % 2][1].wait()

    return k(x2d, e3d, r3d, cnt16)


def _sc_combine_call(outs, pos3, wcat):
    tpw = _NTOK // _NW
    ch = 16
    nch = tpw // ch
    nvr = _D // 16
    mesh = plsc.VectorSubcoreMesh(core_axis_name="c", subcore_axis_name="s")

    @functools.partial(
        pl.kernel,
        out_type=jax.ShapeDtypeStruct((_NTOK, _D), jnp.float32),
        mesh=mesh,
        compiler_params=pltpu.CompilerParams(needs_layout_passes=False),
        scratch_types=[
            pltpu.VMEM((nch // 2, 2 * ch), jnp.int32),
            pltpu.VMEM((nch // 2, 2 * ch), jnp.int32),
            pltpu.VMEM((_K, tpw), jnp.float32),
            pltpu.VMEM((2, ch, _D), jnp.float32),
            pltpu.VMEM((2, ch, _D), jnp.float32),
            pltpu.VMEM((2, ch, _D), jnp.float32),
            [pltpu.SemaphoreType.DMA] * 2,
            [pltpu.SemaphoreType.DMA] * 2,
            [pltpu.SemaphoreType.DMA] * 2,
        ],
    )
    def k(src_hbm, pos_hbm, w_hbm, out_hbm, p1_v, p2_v, w_v, bufa_v, bufb_v,
          bufo_v, asems, bsems, wsems):
        wid = lax.axis_index("s") * 2 + lax.axis_index("c")
        base = wid * tpw
        pltpu.sync_copy(pos_hbm.at[wid], p1_v)
        pltpu.sync_copy(pos_hbm.at[_NW + wid], p2_v)
        pltpu.sync_copy(w_hbm.at[wid], w_v)

        def chunk_idx(p_v, c):
            return p_v[c // 2, pl.ds((c % 2) * ch, ch)]

        def start_gathers(c, s):
            ga = pltpu.async_copy(src_hbm.at[chunk_idx(p1_v, c)],
                                  bufa_v.at[s], asems[s])
            gb = pltpu.async_copy(src_hbm.at[chunk_idx(p2_v, c)],
                                  bufb_v.at[s], bsems[s])
            return ga, gb

        gcps = [start_gathers(0, 0), start_gathers(1, 1)]
        wcps = [None, None]
        for c in range(nch):
            s = c % 2
            gcps[s][0].wait()
            gcps[s][1].wait()
            if wcps[s] is not None:
                wcps[s].wait()

            def row_body(r, carry, c=c, s=s):
                lane0 = jnp.zeros((16,), jnp.int32)
                ridx = lane0 + (c * ch + r)
                wa = plsc.load_gather(w_v, [lane0, ridx])
                wb = plsc.load_gather(w_v, [lane0 + 1, ridx])

                def vec_body(j, carry2):
                    a = bufa_v[s, r, pl.ds(j * 16, 16)]
                    b = bufb_v[s, r, pl.ds(j * 16, 16)]
                    bufo_v[s, r, pl.ds(j * 16, 16)] = wa * a + wb * b
                    return carry2

                lax.fori_loop(0, nvr, vec_body, 0, unroll=4)
                return carry

            lax.fori_loop(0, ch, row_body, 0)
            wcps[s] = pltpu.async_copy(
                bufo_v.at[s], out_hbm.at[pl.ds(base + c * ch, ch)], wsems[s])
            if c + 2 < nch:
                gcps[s] = start_gathers(c + 2, s)
        wcps[0].wait()
        wcps[1].wait()

    return k(outs, pos3, wcat)


def kernel(x, router_W, gate_W, up_W, down_W):
    x2d = x.reshape(_NTOK, _D)
    rwp = jnp.zeros((_EP, _D), jnp.float32).at[:_E].set(router_W)

    e1, e2, w1, w2, r1, r2, cts, loss, nu, tile_e = _router_call(x2d, rwp)

    cnt16 = cts[0].astype(jnp.int32)
    tpw = _NTOK // _NW
    e3d = jnp.concatenate(
        [e1.reshape(_NW, 1, tpw), e2.reshape(_NW, 1, tpw)], axis=1)
    r3d = jnp.concatenate(
        [r1.reshape(_NW, 1, tpw), r2.reshape(_NW, 1, tpw)], axis=1)

    wcat = jnp.concatenate(
        [w1.reshape(_NW, 1, tpw), w2.reshape(_NW, 1, tpw)], axis=1)

    xs, pos3 = _sc_scatter_call(x2d, e3d, r3d, cnt16)
    outs = _mlp_call(tile_e, nu, xs, gate_W, up_W, down_W)
    final = _sc_combine_call(outs, pos3, wcat)
    return final.reshape(x.shape), loss[0, 0]
